# Initial kernel scaffold; baseline (speedup 1.0000x reference)
import jax
import jax.numpy as jnp
from jax.experimental import pallas as pl


def kernel(idx, t, src, dst, event_t, event_feat, nbr_idx, nbr_t, nbr_edge_feat, memory, node_feat, time_w, time_b, W_ih, b_ih, W_hh, b_hh, W_q, W_k, W_v, W_o1, W_o2):
    raise NotImplementedError("write your pallas kernel here")


# speedup vs baseline: 2.8560x; 2.8560x over previous
import functools

import jax
import jax.numpy as jnp
from jax import lax
from jax.experimental import pallas as pl
from jax.experimental.pallas import tpu as pltpu
from jax.experimental.pallas import tpu_sc as plsc

NC, NS, CH = 2, 16, 64

F32 = jnp.float32


def _sc_scatter_mem_body(src_hbm, dst_hbm, mem_hbm, zm_hbm, accm_out,
                         src_idx, dst_idx, rows, acc_m, sem):
    n_pad = acc_m.shape[0]
    e = src_hbm.shape[0]
    c = lax.axis_index("c")
    s = lax.axis_index("s")
    rows_per = n_pad // NS
    r0 = s * rows_per
    pltpu.sync_copy(zm_hbm.at[pl.ds(r0, rows_per)], acc_m.at[pl.ds(r0, rows_per)])
    plsc.subcore_barrier()
    cpc = (e // CH) // NC
    iters = (cpc + NS - 1) // NS

    def body(i, carry):
        j = i * NS + s

        @pl.when(j < cpc)
        def _():
            base = (c * cpc + j) * CH
            pltpu.sync_copy(src_hbm.at[pl.ds(base, CH)], src_idx)
            pltpu.sync_copy(dst_hbm.at[pl.ds(base, CH)], dst_idx)
            pltpu.async_copy(mem_hbm.at[dst_idx], rows, sem).wait()
            pltpu.sync_copy(rows, acc_m.at[src_idx], add=True)
            pltpu.async_copy(mem_hbm.at[src_idx], rows, sem).wait()
            pltpu.sync_copy(rows, acc_m.at[dst_idx], add=True)

        return carry

    lax.fori_loop(0, iters, body, 0)
    plsc.subcore_barrier()
    pltpu.sync_copy(acc_m.at[pl.ds(r0, rows_per)],
                    accm_out.at[c, pl.ds(r0, rows_per)])


def _sc_scatter_pay_body(src_hbm, dst_hbm, pay_hbm, zm_hbm, accs_out,
                         src_idx, dst_idx, pay_buf, acc_s, sem):
    n_pad = acc_s.shape[0]
    e = src_hbm.shape[0]
    c = lax.axis_index("c")
    s = lax.axis_index("s")
    rows_per = n_pad // NS
    r0 = s * rows_per
    pltpu.sync_copy(zm_hbm.at[pl.ds(r0, rows_per)], acc_s.at[pl.ds(r0, rows_per)])
    plsc.subcore_barrier()
    cpc = (e // CH) // NC
    iters = (cpc + NS - 1) // NS

    def body(i, carry):
        j = i * NS + s

        @pl.when(j < cpc)
        def _():
            base = (c * cpc + j) * CH
            pltpu.sync_copy(src_hbm.at[pl.ds(base, CH)], src_idx)
            pltpu.sync_copy(dst_hbm.at[pl.ds(base, CH)], dst_idx)
            pltpu.sync_copy(pay_hbm.at[pl.ds(base, CH)], pay_buf)
            pltpu.sync_copy(pay_buf, acc_s.at[src_idx], add=True)
            pltpu.sync_copy(pay_buf, acc_s.at[dst_idx], add=True)

        return carry

    lax.fori_loop(0, iters, body, 0)
    plsc.subcore_barrier()
    pltpu.sync_copy(acc_s.at[pl.ds(r0, rows_per)],
                    accs_out.at[c, pl.ds(r0, rows_per)])


def _sc_scatter(src, dst, memory, payload):
    n, d = memory.shape
    n_pad = ((n + 8 * NS - 1) // (8 * NS)) * (8 * NS)
    mesh = plsc.VectorSubcoreMesh(core_axis_name="c", subcore_axis_name="s")
    zm = jnp.zeros((n_pad, d), F32)
    fn_m = functools.partial(
        pl.kernel,
        out_type=jax.ShapeDtypeStruct((NC, n_pad, d), F32),
        mesh=mesh,
        scratch_types=[
            pltpu.VMEM((CH,), jnp.int32),
            pltpu.VMEM((CH,), jnp.int32),
            pltpu.VMEM((CH, d), F32),
            pltpu.VMEM_SHARED((n_pad, d), F32),
            pltpu.SemaphoreType.DMA,
        ],
    )(_sc_scatter_mem_body)
    accm = fn_m(src, dst, memory, zm)
    fn_p = functools.partial(
        pl.kernel,
        out_type=jax.ShapeDtypeStruct((NC, n_pad, d), F32),
        mesh=mesh,
        scratch_types=[
            pltpu.VMEM((CH,), jnp.int32),
            pltpu.VMEM((CH,), jnp.int32),
            pltpu.VMEM((CH, d), F32),
            pltpu.VMEM_SHARED((n_pad, d), F32),
            pltpu.SemaphoreType.DMA,
        ],
    )(_sc_scatter_pay_body)
    accs = fn_p(src, dst, payload, zm)
    return accm, accs


def _sc_gather_body(h_hbm, idx_hbm, out_hbm, idxb, rows, sem):
    c = lax.axis_index("c")
    s = lax.axis_index("s")
    w = s * NC + c
    n_chunks = idx_hbm.shape[0] // CH
    iters = n_chunks // (NC * NS)

    def body(i, carry):
        base = (i * NC * NS + w) * CH
        pltpu.sync_copy(idx_hbm.at[pl.ds(base, CH)], idxb)
        pltpu.async_copy(h_hbm.at[idxb], rows, sem).wait()
        pltpu.sync_copy(rows, out_hbm.at[pl.ds(base, CH)])
        return carry

    lax.fori_loop(0, iters, body, 0)


def _sc_gather(h_all, idx_all):
    n, d = h_all.shape
    m = idx_all.shape[0]
    mesh = plsc.VectorSubcoreMesh(core_axis_name="c", subcore_axis_name="s")
    fn = functools.partial(
        pl.kernel,
        out_type=jax.ShapeDtypeStruct((m, d), F32),
        mesh=mesh,
        scratch_types=[
            pltpu.VMEM((CH,), jnp.int32),
            pltpu.VMEM((CH, d), F32),
            pltpu.SemaphoreType.DMA,
        ],
    )(_sc_gather_body)
    return fn(h_all, idx_all)


def _payload_body(ef_ref, et_ref, w_ref, b_ref, out_ref):
    r = ef_ref.shape[0]
    te = jnp.cos(et_ref[...] * w_ref[...] + b_ref[...])
    out_ref[...] = jnp.concatenate(
        [ef_ref[...], te, jnp.ones((r, 1), F32), jnp.zeros((r, 95), F32)],
        axis=1)


def _tc_payload(event_feat, event_t, time_w, time_b):
    e, de = event_feat.shape
    blk = 8000
    grid = e // blk
    return pl.pallas_call(
        _payload_body,
        grid=(grid,),
        in_specs=[
            pl.BlockSpec((blk, de), lambda i: (i, 0)),
            pl.BlockSpec((blk, 1), lambda i: (i, 0)),
            pl.BlockSpec((1, 16), lambda i: (0, 0)),
            pl.BlockSpec((1, 16), lambda i: (0, 0)),
        ],
        out_specs=pl.BlockSpec((blk, 128), lambda i: (i, 0)),
        out_shape=jax.ShapeDtypeStruct((e, 128), F32),
    )(event_feat, event_t.reshape(e, 1), time_w.reshape(1, 16),
      time_b.reshape(1, 16))


def _gru_body(accm_ref, accs_ref, mem_ref, nf_ref, wih_ref, bih_ref, whh_ref,
              bhh_ref, out_ref):
    am = accm_ref[0] + accm_ref[1]
    asml = accs_ref[0] + accs_ref[1]
    cnt = asml[:, 32:33]
    inv = 1.0 / jnp.maximum(cnt, 1.0)
    m = mem_ref[...]
    mm = jnp.concatenate([m, am * inv, asml[:, :32] * inv], axis=1)
    gi = jnp.dot(mm, wih_ref[...], preferred_element_type=F32) + bih_ref[...]
    gh = jnp.dot(m, whh_ref[...], preferred_element_type=F32) + bhh_ref[...]
    d = m.shape[1]
    r = jax.nn.sigmoid(gi[:, :d] + gh[:, :d])
    z = jax.nn.sigmoid(gi[:, d:2 * d] + gh[:, d:2 * d])
    nn = jnp.tanh(gi[:, 2 * d:] + r * gh[:, 2 * d:])
    new_mem = (1.0 - z) * nn + z * m
    out_ref[...] = jnp.where(cnt > 0, new_mem, m) + nf_ref[...]


def _tc_gru(accm, accs, memory, node_feat, w_ih, b_ih, w_hh, b_hh):
    n, d = memory.shape
    blk = 2000
    grid = n // blk
    return pl.pallas_call(
        _gru_body,
        grid=(grid,),
        in_specs=[
            pl.BlockSpec((NC, blk, d), lambda i: (0, i, 0)),
            pl.BlockSpec((NC, blk, d), lambda i: (0, i, 0)),
            pl.BlockSpec((blk, d), lambda i: (i, 0)),
            pl.BlockSpec((blk, d), lambda i: (i, 0)),
            pl.BlockSpec(w_ih.shape, lambda i: (0, 0)),
            pl.BlockSpec((1, 3 * d), lambda i: (0, 0)),
            pl.BlockSpec(w_hh.shape, lambda i: (0, 0)),
            pl.BlockSpec((1, 3 * d), lambda i: (0, 0)),
        ],
        out_specs=pl.BlockSpec((blk, d), lambda i: (i, 0)),
        out_shape=jax.ShapeDtypeStruct((n, d), F32),
    )(accm, accs, memory, node_feat, w_ih, b_ih.reshape(1, -1), w_hh,
      b_hh.reshape(1, -1))


def _attn_body(hs_ref, hn_ref, nef_ref, t_ref, nbt_ref, tw_ref, tb_ref,
               wq_ref, wk_ref, wv_ref, wo1_ref, wo2_ref, out_ref):
    r, d = hs_ref.shape
    k = nbt_ref.shape[1]
    hs = hs_ref[...]
    tw = tw_ref[...].reshape(1, 1, 16)
    tb = tb_ref[...].reshape(1, 1, 16)
    teq = jnp.cos(tb_ref[...])
    wq = wq_ref[...]
    q = (jnp.dot(hs, wq[:d], preferred_element_type=F32)
         + jnp.dot(teq, wq[d:], preferred_element_type=F32))
    dt = t_ref[...] - nbt_ref[...]
    ten = jnp.cos(dt[..., None] * tw + tb).reshape(r * k, 16)
    hn = hn_ref[...]
    nef = nef_ref[...]
    wk, wv = wk_ref[...], wv_ref[...]
    kk = (jnp.dot(hn, wk[:d], preferred_element_type=F32)
          + jnp.dot(nef, wk[d:d + 16], preferred_element_type=F32)
          + jnp.dot(ten, wk[d + 16:], preferred_element_type=F32))
    vv = (jnp.dot(hn, wv[:d], preferred_element_type=F32)
          + jnp.dot(nef, wv[d:d + 16], preferred_element_type=F32)
          + jnp.dot(ten, wv[d + 16:], preferred_element_type=F32))
    logits = jnp.sum(q[:, None, :] * kk.reshape(r, k, d), axis=-1)
    logits = logits * (1.0 / jnp.sqrt(jnp.float32(d)))
    mx = jnp.max(logits, axis=-1, keepdims=True)
    ex = jnp.exp(logits - mx)
    attn = ex / jnp.sum(ex, axis=-1, keepdims=True)
    out = jnp.sum(attn[..., None] * vv.reshape(r, k, d), axis=1)
    wo1 = wo1_ref[...]
    hcat = (jnp.dot(out, wo1[:d], preferred_element_type=F32)
            + jnp.dot(hs, wo1[d:], preferred_element_type=F32))
    emb = jnp.dot(jax.nn.relu(hcat), wo2_ref[...],
                  preferred_element_type=F32)
    out_ref[...] = emb


def _tc_attn(h_src, h_nbr_flat, nbr_ef_flat, t, nbr_t, time_w, time_b,
             w_q, w_k, w_v, w_o1, w_o2):
    b, d = h_src.shape
    k = nbr_t.shape[1]
    blk = 512
    grid = b // blk
    return pl.pallas_call(
        _attn_body,
        grid=(grid,),
        in_specs=[
            pl.BlockSpec((blk, d), lambda i: (i, 0)),
            pl.BlockSpec((blk * k, d), lambda i: (i, 0)),
            pl.BlockSpec((blk * k, 16), lambda i: (i, 0)),
            pl.BlockSpec((blk, 1), lambda i: (i, 0)),
            pl.BlockSpec((blk, k), lambda i: (i, 0)),
            pl.BlockSpec((1, 16), lambda i: (0, 0)),
            pl.BlockSpec((1, 16), lambda i: (0, 0)),
            pl.BlockSpec(w_q.shape, lambda i: (0, 0)),
            pl.BlockSpec(w_k.shape, lambda i: (0, 0)),
            pl.BlockSpec(w_v.shape, lambda i: (0, 0)),
            pl.BlockSpec(w_o1.shape, lambda i: (0, 0)),
            pl.BlockSpec(w_o2.shape, lambda i: (0, 0)),
        ],
        out_specs=pl.BlockSpec((blk, d), lambda i: (i, 0)),
        out_shape=jax.ShapeDtypeStruct((b, d), F32),
    )(h_src, h_nbr_flat, nbr_ef_flat, t.reshape(b, 1), nbr_t,
      time_w.reshape(1, 16), time_b.reshape(1, 16), w_q, w_k, w_v, w_o1, w_o2)


def kernel(idx, t, src, dst, event_t, event_feat, nbr_idx, nbr_t,
           nbr_edge_feat, memory, node_feat, time_w, time_b, W_ih, b_ih,
           W_hh, b_hh, W_q, W_k, W_v, W_o1, W_o2):
    b, k = nbr_idx.shape
    payload = _tc_payload(event_feat, event_t, time_w, time_b)
    accm, accs = _sc_scatter(src, dst, memory, payload)
    h_all = _tc_gru(accm, accs, memory, node_feat, W_ih, b_ih, W_hh, b_hh)
    idx_all = jnp.concatenate([idx, nbr_idx.reshape(-1)])
    rows = _sc_gather(h_all, idx_all)
    h_src = rows[:b]
    h_nbr_flat = rows[b:]
    emb = _tc_attn(h_src, h_nbr_flat, nbr_edge_feat.reshape(b * k, 16),
                   t, nbr_t, time_w, time_b, W_q, W_k, W_v, W_o1, W_o2)
    return emb


# speedup vs baseline: 3.2851x; 1.1502x over previous
import functools

import jax
import jax.numpy as jnp
from jax import lax
from jax.experimental import pallas as pl
from jax.experimental.pallas import tpu as pltpu
from jax.experimental.pallas import tpu_sc as plsc

NC, NS, CH = 2, 16, 128

F32 = jnp.float32


def _sc_scatter_mem_body(src_hbm, dst_hbm, mem_hbm, zm_hbm, accm_out,
                         src_idx, dst_idx, rows, acc_m, sem):
    n_pad = acc_m.shape[0]
    e = src_hbm.shape[0]
    c = lax.axis_index("c")
    s = lax.axis_index("s")
    rows_per = n_pad // NS
    r0 = s * rows_per
    pltpu.sync_copy(zm_hbm.at[pl.ds(r0, rows_per)], acc_m.at[pl.ds(r0, rows_per)])
    plsc.subcore_barrier()
    cpc = (e // CH) // NC
    iters = (cpc + NS - 1) // NS

    def body(i, carry):
        j = i * NS + s

        @pl.when(j < cpc)
        def _():
            base = (c * cpc + j) * CH
            pltpu.sync_copy(src_hbm.at[pl.ds(base, CH)], src_idx)
            pltpu.sync_copy(dst_hbm.at[pl.ds(base, CH)], dst_idx)
            pltpu.async_copy(mem_hbm.at[dst_idx], rows, sem).wait()
            pltpu.sync_copy(rows, acc_m.at[src_idx], add=True)
            pltpu.async_copy(mem_hbm.at[src_idx], rows, sem).wait()
            pltpu.sync_copy(rows, acc_m.at[dst_idx], add=True)

        return carry

    lax.fori_loop(0, iters, body, 0)
    plsc.subcore_barrier()
    pltpu.sync_copy(acc_m.at[pl.ds(r0, rows_per)],
                    accm_out.at[c, pl.ds(r0, rows_per)])


def _sc_scatter_pay_body(src_hbm, dst_hbm, pay_hbm, zm_hbm, accs_out,
                         src_idx, dst_idx, pay_buf, acc_s, sem):
    n_pad = acc_s.shape[0]
    e = src_hbm.shape[0]
    c = lax.axis_index("c")
    s = lax.axis_index("s")
    rows_per = n_pad // NS
    r0 = s * rows_per
    pltpu.sync_copy(zm_hbm.at[pl.ds(r0, rows_per)], acc_s.at[pl.ds(r0, rows_per)])
    plsc.subcore_barrier()
    cpc = (e // CH) // NC
    iters = (cpc + NS - 1) // NS

    def body(i, carry):
        j = i * NS + s

        @pl.when(j < cpc)
        def _():
            base = (c * cpc + j) * CH
            pltpu.sync_copy(src_hbm.at[pl.ds(base, CH)], src_idx)
            pltpu.sync_copy(dst_hbm.at[pl.ds(base, CH)], dst_idx)
            pltpu.sync_copy(pay_hbm.at[pl.ds(base, CH)], pay_buf)
            pltpu.sync_copy(pay_buf, acc_s.at[src_idx], add=True)
            pltpu.sync_copy(pay_buf, acc_s.at[dst_idx], add=True)

        return carry

    lax.fori_loop(0, iters, body, 0)
    plsc.subcore_barrier()
    pltpu.sync_copy(acc_s.at[pl.ds(r0, rows_per)],
                    accs_out.at[c, pl.ds(r0, rows_per)])


def _sc_scatter_call(body, src, dst, table, n, d):
    n_pad = ((n + 8 * NS - 1) // (8 * NS)) * (8 * NS)
    mesh = plsc.VectorSubcoreMesh(core_axis_name="c", subcore_axis_name="s")
    zm = jnp.zeros((n_pad, d), F32)
    fn = functools.partial(
        pl.kernel,
        out_type=jax.ShapeDtypeStruct((NC, n_pad, d), F32),
        mesh=mesh,
        scratch_types=[
            pltpu.VMEM((CH,), jnp.int32),
            pltpu.VMEM((CH,), jnp.int32),
            pltpu.VMEM((CH, d), F32),
            pltpu.VMEM_SHARED((n_pad, d), F32),
            pltpu.SemaphoreType.DMA,
        ],
    )(body)
    return fn(src, dst, table, zm)


def _sc_gather_body(h_hbm, idx_hbm, out_hbm, idxb, rows, sem):
    c = lax.axis_index("c")
    s = lax.axis_index("s")
    w = s * NC + c
    n_chunks = idx_hbm.shape[0] // CH
    iters = n_chunks // (NC * NS)

    def body(i, carry):
        base = (i * NC * NS + w) * CH
        pltpu.sync_copy(idx_hbm.at[pl.ds(base, CH)], idxb)
        pltpu.async_copy(h_hbm.at[idxb], rows, sem).wait()
        pltpu.sync_copy(rows, out_hbm.at[pl.ds(base, CH)])
        return carry

    lax.fori_loop(0, iters, body, 0)


def _sc_gather(h_all, idx_all):
    n, d = h_all.shape
    m = idx_all.shape[0]
    mesh = plsc.VectorSubcoreMesh(core_axis_name="c", subcore_axis_name="s")
    fn = functools.partial(
        pl.kernel,
        out_type=jax.ShapeDtypeStruct((m, d), F32),
        mesh=mesh,
        scratch_types=[
            pltpu.VMEM((CH,), jnp.int32),
            pltpu.VMEM((CH, d), F32),
            pltpu.SemaphoreType.DMA,
        ],
    )(_sc_gather_body)
    return fn(h_all, idx_all)


def _payload_body(ef_ref, et_ref, w_ref, b_ref, out_ref):
    r = ef_ref.shape[0]
    te = jnp.cos(et_ref[...] * w_ref[...] + b_ref[...])
    out_ref[...] = jnp.concatenate(
        [ef_ref[...], te, jnp.ones((r, 1), F32), jnp.zeros((r, 95), F32)],
        axis=1)


def _tc_payload(event_feat, event_t16, time_w, time_b):
    e, de = event_feat.shape
    blk = 8000
    grid = e // blk
    return pl.pallas_call(
        _payload_body,
        grid=(grid,),
        in_specs=[
            pl.BlockSpec((blk, de), lambda i: (i, 0)),
            pl.BlockSpec((blk, 16), lambda i: (i, 0)),
            pl.BlockSpec((1, 16), lambda i: (0, 0)),
            pl.BlockSpec((1, 16), lambda i: (0, 0)),
        ],
        out_specs=pl.BlockSpec((blk, 128), lambda i: (i, 0)),
        out_shape=jax.ShapeDtypeStruct((e, 128), F32),
    )(event_feat, event_t16, time_w.reshape(1, 16),
      time_b.reshape(1, 16))


def _gru_body(accm_ref, accs_ref, mem_ref, nf_ref, wih_ref, bih_ref, whh_ref,
              bhh_ref, out_ref):
    am = accm_ref[0] + accm_ref[1]
    asml = accs_ref[0] + accs_ref[1]
    cnt = asml[:, 32:33]
    inv = 1.0 / jnp.maximum(cnt, 1.0)
    m = mem_ref[...]
    mm = jnp.concatenate([m, am * inv, asml[:, :32] * inv], axis=1)
    gi = jnp.dot(mm, wih_ref[...], preferred_element_type=F32) + bih_ref[...]
    gh = jnp.dot(m, whh_ref[...], preferred_element_type=F32) + bhh_ref[...]
    d = m.shape[1]
    r = jax.nn.sigmoid(gi[:, :d] + gh[:, :d])
    z = jax.nn.sigmoid(gi[:, d:2 * d] + gh[:, d:2 * d])
    nn = jnp.tanh(gi[:, 2 * d:] + r * gh[:, 2 * d:])
    new_mem = (1.0 - z) * nn + z * m
    out_ref[...] = jnp.where(cnt > 0, new_mem, m) + nf_ref[...]


def _tc_gru(accm, accs, memory, node_feat, w_ih, b_ih, w_hh, b_hh):
    n, d = memory.shape
    blk = 2000
    grid = n // blk
    return pl.pallas_call(
        _gru_body,
        grid=(grid,),
        in_specs=[
            pl.BlockSpec((NC, blk, d), lambda i: (0, i, 0)),
            pl.BlockSpec((NC, blk, d), lambda i: (0, i, 0)),
            pl.BlockSpec((blk, d), lambda i: (i, 0)),
            pl.BlockSpec((blk, d), lambda i: (i, 0)),
            pl.BlockSpec(w_ih.shape, lambda i: (0, 0)),
            pl.BlockSpec((1, 3 * d), lambda i: (0, 0)),
            pl.BlockSpec(w_hh.shape, lambda i: (0, 0)),
            pl.BlockSpec((1, 3 * d), lambda i: (0, 0)),
        ],
        out_specs=pl.BlockSpec((blk, d), lambda i: (i, 0)),
        out_shape=jax.ShapeDtypeStruct((n, d), F32),
    )(accm, accs, memory, node_feat, w_ih, b_ih.reshape(1, -1), w_hh,
      b_hh.reshape(1, -1))


def _attn_body(hs_ref, hn_ref, nef_ref, t_ref, nbt_ref, tw_ref, tb_ref,
               wq_ref, wk_ref, wv_ref, wo1_ref, wo2_ref, out_ref):
    r, d = hs_ref.shape
    k = nbt_ref.shape[1]
    hs = hs_ref[...]
    tw = tw_ref[...].reshape(1, 1, 16)
    tb = tb_ref[...].reshape(1, 1, 16)
    teq = jnp.cos(tb_ref[...])
    wq = wq_ref[...]
    q = (jnp.dot(hs, wq[:d], preferred_element_type=F32)
         + jnp.dot(teq, wq[d:], preferred_element_type=F32))
    dt = t_ref[...] - nbt_ref[...]
    ten = jnp.cos(dt[..., None] * tw + tb).reshape(r * k, 16)
    hn = hn_ref[...]
    nef = nef_ref[...]
    wk, wv = wk_ref[...], wv_ref[...]
    kk = (jnp.dot(hn, wk[:d], preferred_element_type=F32)
          + jnp.dot(nef, wk[d:d + 16], preferred_element_type=F32)
          + jnp.dot(ten, wk[d + 16:], preferred_element_type=F32))
    vv = (jnp.dot(hn, wv[:d], preferred_element_type=F32)
          + jnp.dot(nef, wv[d:d + 16], preferred_element_type=F32)
          + jnp.dot(ten, wv[d + 16:], preferred_element_type=F32))
    logits = jnp.sum(q[:, None, :] * kk.reshape(r, k, d), axis=-1)
    logits = logits * (1.0 / jnp.sqrt(jnp.float32(d)))
    mx = jnp.max(logits, axis=-1, keepdims=True)
    ex = jnp.exp(logits - mx)
    attn = ex / jnp.sum(ex, axis=-1, keepdims=True)
    out = jnp.sum(attn[..., None] * vv.reshape(r, k, d), axis=1)
    wo1 = wo1_ref[...]
    hcat = (jnp.dot(out, wo1[:d], preferred_element_type=F32)
            + jnp.dot(hs, wo1[d:], preferred_element_type=F32))
    emb = jnp.dot(jax.nn.relu(hcat), wo2_ref[...],
                  preferred_element_type=F32)
    out_ref[...] = emb


def _tc_attn(h_src, h_nbr_flat, nbr_ef_flat, t, nbr_t, time_w, time_b,
             w_q, w_k, w_v, w_o1, w_o2):
    b, d = h_src.shape
    k = nbr_t.shape[1]
    blk = 512
    grid = b // blk
    return pl.pallas_call(
        _attn_body,
        grid=(grid,),
        in_specs=[
            pl.BlockSpec((blk, d), lambda i: (i, 0)),
            pl.BlockSpec((blk * k, d), lambda i: (i, 0)),
            pl.BlockSpec((blk * k, 16), lambda i: (i, 0)),
            pl.BlockSpec((blk, k), lambda i: (i, 0)),
            pl.BlockSpec((blk, k), lambda i: (i, 0)),
            pl.BlockSpec((1, 16), lambda i: (0, 0)),
            pl.BlockSpec((1, 16), lambda i: (0, 0)),
            pl.BlockSpec(w_q.shape, lambda i: (0, 0)),
            pl.BlockSpec(w_k.shape, lambda i: (0, 0)),
            pl.BlockSpec(w_v.shape, lambda i: (0, 0)),
            pl.BlockSpec(w_o1.shape, lambda i: (0, 0)),
            pl.BlockSpec(w_o2.shape, lambda i: (0, 0)),
        ],
        out_specs=pl.BlockSpec((blk, d), lambda i: (i, 0)),
        out_shape=jax.ShapeDtypeStruct((b, d), F32),
    )(h_src, h_nbr_flat, nbr_ef_flat, jnp.broadcast_to(t[:, None], (b, k)),
      nbr_t, time_w.reshape(1, 16), time_b.reshape(1, 16), w_q, w_k, w_v,
      w_o1, w_o2)


def kernel(idx, t, src, dst, event_t, event_feat, nbr_idx, nbr_t,
           nbr_edge_feat, memory, node_feat, time_w, time_b, W_ih, b_ih,
           W_hh, b_hh, W_q, W_k, W_v, W_o1, W_o2):
    b, k = nbr_idx.shape
    n, d = memory.shape
    e = src.shape[0]
    accm = _sc_scatter_call(_sc_scatter_mem_body, src, dst, memory, n, d)
    et16 = jnp.broadcast_to(event_t[:, None], (e, 16))
    payload = _tc_payload(event_feat, et16, time_w, time_b)
    accs = _sc_scatter_call(_sc_scatter_pay_body, src, dst, payload, n, d)
    h_all = _tc_gru(accm, accs, memory, node_feat, W_ih, b_ih, W_hh, b_hh)
    idx_all = jnp.concatenate([idx, nbr_idx.reshape(-1)])
    rows = _sc_gather(h_all, idx_all)
    h_src = rows[:b]
    h_nbr_flat = rows[b:]
    emb = _tc_attn(h_src, h_nbr_flat, nbr_edge_feat.reshape(b * k, 16),
                   t, nbr_t, time_w, time_b, W_q, W_k, W_v, W_o1, W_o2)
    return emb


# speedup vs baseline: 3.7343x; 1.1367x over previous
import functools

import jax
import jax.numpy as jnp
from jax import lax
from jax.experimental import pallas as pl
from jax.experimental.pallas import tpu as pltpu
from jax.experimental.pallas import tpu_sc as plsc

NC, NS = 2, 16
CH = 128
CHM = 64

F32 = jnp.float32


def _sc_scatter_mem_body(src_hbm, dst_hbm, mem_hbm, zm_hbm, accm_out,
                         src_idx, dst_idx, rows_s, rows_d,
                         sem_s0, sem_s1, sem_d0, sem_d1, acc_m):
    n_pad = acc_m.shape[0]
    e = src_hbm.shape[0]
    c = lax.axis_index("c")
    s = lax.axis_index("s")
    rows_per = n_pad // NS
    r0 = s * rows_per
    pltpu.sync_copy(zm_hbm.at[pl.ds(r0, rows_per)], acc_m.at[pl.ds(r0, rows_per)])
    plsc.subcore_barrier()
    cpc = (e // CHM) // NC
    iters = (cpc + NS - 1) // NS
    sem_s = (sem_s0, sem_s1)
    sem_d = (sem_d0, sem_d1)

    def issue(i, slot):
        j = i * NS + s

        @pl.when(j < cpc)
        def _():
            base = (c * cpc + j) * CHM
            pltpu.sync_copy(src_hbm.at[pl.ds(base, CHM)], src_idx.at[slot])
            pltpu.sync_copy(dst_hbm.at[pl.ds(base, CHM)], dst_idx.at[slot])
            pltpu.async_copy(mem_hbm.at[dst_idx.at[slot]], rows_d.at[slot],
                             sem_d[slot])
            pltpu.async_copy(mem_hbm.at[src_idx.at[slot]], rows_s.at[slot],
                             sem_s[slot])

    def drain(i, slot):
        j = i * NS + s

        @pl.when(j < cpc)
        def _():
            pltpu.make_async_copy(mem_hbm.at[dst_idx.at[slot]],
                                  rows_d.at[slot], sem_d[slot]).wait()
            pltpu.sync_copy(rows_d.at[slot], acc_m.at[src_idx.at[slot]],
                            add=True)
            pltpu.make_async_copy(mem_hbm.at[src_idx.at[slot]],
                                  rows_s.at[slot], sem_s[slot]).wait()
            pltpu.sync_copy(rows_s.at[slot], acc_m.at[dst_idx.at[slot]],
                            add=True)

    issue(0, 0)

    def body(i2, carry):
        for b2 in (0, 1):
            i = i2 * 2 + b2
            issue(i + 1, 1 - b2)
            drain(i, b2)
        return carry

    lax.fori_loop(0, (iters + 1) // 2, body, 0)
    plsc.subcore_barrier()
    pltpu.sync_copy(acc_m.at[pl.ds(r0, rows_per)],
                    accm_out.at[c, pl.ds(r0, rows_per)])


def _sc_scatter_pay_body(src_hbm, dst_hbm, pay_hbm, zm_hbm, accs_out,
                         src_idx, dst_idx, pay_buf, sem0, sem1, acc_s):
    n_pad = acc_s.shape[0]
    e = src_hbm.shape[0]
    c = lax.axis_index("c")
    s = lax.axis_index("s")
    rows_per = n_pad // NS
    r0 = s * rows_per
    pltpu.sync_copy(zm_hbm.at[pl.ds(r0, rows_per)], acc_s.at[pl.ds(r0, rows_per)])
    plsc.subcore_barrier()
    cpc = (e // CH) // NC
    iters = (cpc + NS - 1) // NS
    sems = (sem0, sem1)

    def issue(i, slot):
        j = i * NS + s

        @pl.when(j < cpc)
        def _():
            base = (c * cpc + j) * CH
            pltpu.sync_copy(src_hbm.at[pl.ds(base, CH)], src_idx.at[slot])
            pltpu.sync_copy(dst_hbm.at[pl.ds(base, CH)], dst_idx.at[slot])
            pltpu.async_copy(pay_hbm.at[pl.ds(base, CH)], pay_buf.at[slot],
                             sems[slot])

    def drain(i, slot):
        j = i * NS + s

        @pl.when(j < cpc)
        def _():
            base = (c * cpc + j) * CH
            pltpu.make_async_copy(pay_hbm.at[pl.ds(base, CH)],
                                  pay_buf.at[slot], sems[slot]).wait()
            pltpu.sync_copy(pay_buf.at[slot], acc_s.at[src_idx.at[slot]],
                            add=True)
            pltpu.sync_copy(pay_buf.at[slot], acc_s.at[dst_idx.at[slot]],
                            add=True)

    issue(0, 0)

    def body(i2, carry):
        for b2 in (0, 1):
            i = i2 * 2 + b2
            issue(i + 1, 1 - b2)
            drain(i, b2)
        return carry

    lax.fori_loop(0, (iters + 1) // 2, body, 0)
    plsc.subcore_barrier()
    pltpu.sync_copy(acc_s.at[pl.ds(r0, rows_per)],
                    accs_out.at[c, pl.ds(r0, rows_per)])


def _sc_scatter_mem(src, dst, memory, n, d):
    n_pad = ((n + 8 * NS - 1) // (8 * NS)) * (8 * NS)
    mesh = plsc.VectorSubcoreMesh(core_axis_name="c", subcore_axis_name="s")
    zm = jnp.zeros((n_pad, d), F32)
    fn = functools.partial(
        pl.kernel,
        out_type=jax.ShapeDtypeStruct((NC, n_pad, d), F32),
        mesh=mesh,
        scratch_types=[
            pltpu.VMEM((2, CHM), jnp.int32),
            pltpu.VMEM((2, CHM), jnp.int32),
            pltpu.VMEM((2, CHM, d), F32),
            pltpu.VMEM((2, CHM, d), F32),
            pltpu.SemaphoreType.DMA,
            pltpu.SemaphoreType.DMA,
            pltpu.SemaphoreType.DMA,
            pltpu.SemaphoreType.DMA,
            pltpu.VMEM_SHARED((n_pad, d), F32),
        ],
    )(_sc_scatter_mem_body)
    return fn(src, dst, memory, zm)


def _sc_scatter_pay(src, dst, payload, n, d):
    n_pad = ((n + 8 * NS - 1) // (8 * NS)) * (8 * NS)
    mesh = plsc.VectorSubcoreMesh(core_axis_name="c", subcore_axis_name="s")
    zm = jnp.zeros((n_pad, d), F32)
    fn = functools.partial(
        pl.kernel,
        out_type=jax.ShapeDtypeStruct((NC, n_pad, d), F32),
        mesh=mesh,
        scratch_types=[
            pltpu.VMEM((2, CH), jnp.int32),
            pltpu.VMEM((2, CH), jnp.int32),
            pltpu.VMEM((2, CH, d), F32),
            pltpu.SemaphoreType.DMA,
            pltpu.SemaphoreType.DMA,
            pltpu.VMEM_SHARED((n_pad, d), F32),
        ],
    )(_sc_scatter_pay_body)
    return fn(src, dst, payload, zm)


def _sc_gather_body(h_hbm, idx_hbm, out_hbm, idxb, rows, sem0, sem1):
    c = lax.axis_index("c")
    s = lax.axis_index("s")
    w = s * NC + c
    n_chunks = idx_hbm.shape[0] // CH
    iters = n_chunks // (NC * NS)
    sems = (sem0, sem1)

    def issue(i, slot):
        @pl.when(i < iters)
        def _():
            base = (i * NC * NS + w) * CH
            pltpu.sync_copy(idx_hbm.at[pl.ds(base, CH)], idxb.at[slot])
            pltpu.async_copy(h_hbm.at[idxb.at[slot]], rows.at[slot],
                             sems[slot])

    def drain(i, slot):
        @pl.when(i < iters)
        def _():
            base = (i * NC * NS + w) * CH
            pltpu.make_async_copy(h_hbm.at[idxb.at[slot]], rows.at[slot],
                                  sems[slot]).wait()
            pltpu.sync_copy(rows.at[slot], out_hbm.at[pl.ds(base, CH)])

    issue(0, 0)

    def body(i2, carry):
        for b2 in (0, 1):
            i = i2 * 2 + b2
            issue(i + 1, 1 - b2)
            drain(i, b2)
        return carry

    lax.fori_loop(0, (iters + 1) // 2, body, 0)


def _sc_gather(h_all, idx_all):
    n, d = h_all.shape
    m = idx_all.shape[0]
    mesh = plsc.VectorSubcoreMesh(core_axis_name="c", subcore_axis_name="s")
    fn = functools.partial(
        pl.kernel,
        out_type=jax.ShapeDtypeStruct((m, d), F32),
        mesh=mesh,
        scratch_types=[
            pltpu.VMEM((2, CH), jnp.int32),
            pltpu.VMEM((2, CH, d), F32),
            pltpu.SemaphoreType.DMA,
            pltpu.SemaphoreType.DMA,
        ],
    )(_sc_gather_body)
    return fn(h_all, idx_all)


def _payload_body(ef_ref, et_ref, w_ref, b_ref, out_ref):
    r = ef_ref.shape[0]
    te = jnp.cos(et_ref[...] * w_ref[...] + b_ref[...])
    out_ref[...] = jnp.concatenate(
        [ef_ref[...], te, jnp.ones((r, 1), F32), jnp.zeros((r, 95), F32)],
        axis=1)


def _tc_payload(event_feat, event_t16, time_w, time_b):
    e, de = event_feat.shape
    blk = 8000
    grid = e // blk
    return pl.pallas_call(
        _payload_body,
        grid=(grid,),
        in_specs=[
            pl.BlockSpec((blk, de), lambda i: (i, 0)),
            pl.BlockSpec((blk, 16), lambda i: (i, 0)),
            pl.BlockSpec((1, 16), lambda i: (0, 0)),
            pl.BlockSpec((1, 16), lambda i: (0, 0)),
        ],
        out_specs=pl.BlockSpec((blk, 128), lambda i: (i, 0)),
        out_shape=jax.ShapeDtypeStruct((e, 128), F32),
    )(event_feat, event_t16, time_w.reshape(1, 16),
      time_b.reshape(1, 16))


def _gru_body(accm_ref, accs_ref, mem_ref, nf_ref, wih_ref, bih_ref, whh_ref,
              bhh_ref, out_ref):
    am = accm_ref[0] + accm_ref[1]
    asml = accs_ref[0] + accs_ref[1]
    cnt = asml[:, 32:33]
    inv = 1.0 / jnp.maximum(cnt, 1.0)
    m = mem_ref[...]
    mm = jnp.concatenate([m, am * inv, asml[:, :32] * inv], axis=1)
    gi = jnp.dot(mm, wih_ref[...], preferred_element_type=F32) + bih_ref[...]
    gh = jnp.dot(m, whh_ref[...], preferred_element_type=F32) + bhh_ref[...]
    d = m.shape[1]
    r = jax.nn.sigmoid(gi[:, :d] + gh[:, :d])
    z = jax.nn.sigmoid(gi[:, d:2 * d] + gh[:, d:2 * d])
    nn = jnp.tanh(gi[:, 2 * d:] + r * gh[:, 2 * d:])
    new_mem = (1.0 - z) * nn + z * m
    out_ref[...] = jnp.where(cnt > 0, new_mem, m) + nf_ref[...]


def _tc_gru(accm, accs, memory, node_feat, w_ih, b_ih, w_hh, b_hh):
    n, d = memory.shape
    blk = 2000
    grid = n // blk
    return pl.pallas_call(
        _gru_body,
        grid=(grid,),
        in_specs=[
            pl.BlockSpec((NC, blk, d), lambda i: (0, i, 0)),
            pl.BlockSpec((NC, blk, d), lambda i: (0, i, 0)),
            pl.BlockSpec((blk, d), lambda i: (i, 0)),
            pl.BlockSpec((blk, d), lambda i: (i, 0)),
            pl.BlockSpec(w_ih.shape, lambda i: (0, 0)),
            pl.BlockSpec((1, 3 * d), lambda i: (0, 0)),
            pl.BlockSpec(w_hh.shape, lambda i: (0, 0)),
            pl.BlockSpec((1, 3 * d), lambda i: (0, 0)),
        ],
        out_specs=pl.BlockSpec((blk, d), lambda i: (i, 0)),
        out_shape=jax.ShapeDtypeStruct((n, d), F32),
    )(accm, accs, memory, node_feat, w_ih, b_ih.reshape(1, -1), w_hh,
      b_hh.reshape(1, -1))


def _attn_body(hs_ref, hn_ref, nef_ref, t_ref, nbt_ref, tw_ref, tb_ref,
               wq_ref, wk_ref, wv_ref, wo1_ref, wo2_ref, out_ref):
    r, d = hs_ref.shape
    k = nbt_ref.shape[1]
    hs = hs_ref[...]
    tw = tw_ref[...].reshape(1, 1, 16)
    tb = tb_ref[...].reshape(1, 1, 16)
    teq = jnp.cos(tb_ref[...])
    wq = wq_ref[...]
    q = (jnp.dot(hs, wq[:d], preferred_element_type=F32)
         + jnp.dot(teq, wq[d:], preferred_element_type=F32))
    dt = t_ref[...] - nbt_ref[...]
    ten = jnp.cos(dt[..., None] * tw + tb).reshape(r * k, 16)
    hn = hn_ref[...]
    nef = nef_ref[...]
    wk, wv = wk_ref[...], wv_ref[...]
    kk = (jnp.dot(hn, wk[:d], preferred_element_type=F32)
          + jnp.dot(nef, wk[d:d + 16], preferred_element_type=F32)
          + jnp.dot(ten, wk[d + 16:], preferred_element_type=F32))
    vv = (jnp.dot(hn, wv[:d], preferred_element_type=F32)
          + jnp.dot(nef, wv[d:d + 16], preferred_element_type=F32)
          + jnp.dot(ten, wv[d + 16:], preferred_element_type=F32))
    logits = jnp.sum(q[:, None, :] * kk.reshape(r, k, d), axis=-1)
    logits = logits * (1.0 / jnp.sqrt(jnp.float32(d)))
    mx = jnp.max(logits, axis=-1, keepdims=True)
    ex = jnp.exp(logits - mx)
    attn = ex / jnp.sum(ex, axis=-1, keepdims=True)
    out = jnp.sum(attn[..., None] * vv.reshape(r, k, d), axis=1)
    wo1 = wo1_ref[...]
    hcat = (jnp.dot(out, wo1[:d], preferred_element_type=F32)
            + jnp.dot(hs, wo1[d:], preferred_element_type=F32))
    emb = jnp.dot(jax.nn.relu(hcat), wo2_ref[...],
                  preferred_element_type=F32)
    out_ref[...] = emb


def _tc_attn(h_src, h_nbr_flat, nbr_ef_flat, t, nbr_t, time_w, time_b,
             w_q, w_k, w_v, w_o1, w_o2):
    b, d = h_src.shape
    k = nbr_t.shape[1]
    blk = 512
    grid = b // blk
    return pl.pallas_call(
        _attn_body,
        grid=(grid,),
        in_specs=[
            pl.BlockSpec((blk, d), lambda i: (i, 0)),
            pl.BlockSpec((blk * k, d), lambda i: (i, 0)),
            pl.BlockSpec((blk * k, 16), lambda i: (i, 0)),
            pl.BlockSpec((blk, k), lambda i: (i, 0)),
            pl.BlockSpec((blk, k), lambda i: (i, 0)),
            pl.BlockSpec((1, 16), lambda i: (0, 0)),
            pl.BlockSpec((1, 16), lambda i: (0, 0)),
            pl.BlockSpec(w_q.shape, lambda i: (0, 0)),
            pl.BlockSpec(w_k.shape, lambda i: (0, 0)),
            pl.BlockSpec(w_v.shape, lambda i: (0, 0)),
            pl.BlockSpec(w_o1.shape, lambda i: (0, 0)),
            pl.BlockSpec(w_o2.shape, lambda i: (0, 0)),
        ],
        out_specs=pl.BlockSpec((blk, d), lambda i: (i, 0)),
        out_shape=jax.ShapeDtypeStruct((b, d), F32),
    )(h_src, h_nbr_flat, nbr_ef_flat, jnp.broadcast_to(t[:, None], (b, k)),
      nbr_t, time_w.reshape(1, 16), time_b.reshape(1, 16), w_q, w_k, w_v,
      w_o1, w_o2)


def kernel(idx, t, src, dst, event_t, event_feat, nbr_idx, nbr_t,
           nbr_edge_feat, memory, node_feat, time_w, time_b, W_ih, b_ih,
           W_hh, b_hh, W_q, W_k, W_v, W_o1, W_o2):
    b, k = nbr_idx.shape
    n, d = memory.shape
    e = src.shape[0]
    accm = _sc_scatter_mem(src, dst, memory, n, d)
    et16 = jnp.broadcast_to(event_t[:, None], (e, 16))
    payload = _tc_payload(event_feat, et16, time_w, time_b)
    accs = _sc_scatter_pay(src, dst, payload, n, d)
    h_all = _tc_gru(accm, accs, memory, node_feat, W_ih, b_ih, W_hh, b_hh)
    idx_all = jnp.concatenate([idx, nbr_idx.reshape(-1)])
    rows = _sc_gather(h_all, idx_all)
    h_src = rows[:b]
    h_nbr_flat = rows[b:]
    emb = _tc_attn(h_src, h_nbr_flat, nbr_edge_feat.reshape(b * k, 16),
                   t, nbr_t, time_w, time_b, W_q, W_k, W_v, W_o1, W_o2)
    return emb


# speedup vs baseline: 5.4096x; 1.4486x over previous
import functools

import jax
import jax.numpy as jnp
from jax import lax
from jax.experimental import pallas as pl
from jax.experimental.pallas import tpu as pltpu
from jax.experimental.pallas import tpu_sc as plsc

NC, NS = 2, 16
CH = 128
CHM = 64

F32 = jnp.float32

_COS_C = (0.999999999708885, -19.739208718081116, 64.9393881176947,
          -85.45664332355348, 60.24201925362776, -26.404267783730656,
          7.799566579651471, -1.4530463585373052)


def _fast_cos(x):
    u = x * 0.15915494309189535
    u = u - jnp.floor(u + 0.5)
    v = u * u
    p = jnp.float32(_COS_C[7])
    for cc in _COS_C[6::-1]:
        p = p * v + jnp.float32(cc)
    return p


def _sc_scatter_mem_body(src_hbm, dst_hbm, mem_hbm, zm_hbm, accm_out,
                         src_idx, dst_idx, rows_s, rows_d,
                         sem_s0, sem_s1, sem_d0, sem_d1, acc_m):
    n_pad = acc_m.shape[0]
    e = src_hbm.shape[0]
    c = lax.axis_index("c")
    s = lax.axis_index("s")
    rows_per = n_pad // NS
    r0 = s * rows_per
    pltpu.sync_copy(zm_hbm.at[pl.ds(r0, rows_per)], acc_m.at[pl.ds(r0, rows_per)])
    plsc.subcore_barrier()
    cpc = (e // CHM) // NC
    iters = (cpc + NS - 1) // NS
    sem_s = (sem_s0, sem_s1)
    sem_d = (sem_d0, sem_d1)

    def issue(i, slot):
        j = i * NS + s

        @pl.when(j < cpc)
        def _():
            base = (c * cpc + j) * CHM
            pltpu.sync_copy(src_hbm.at[pl.ds(base, CHM)], src_idx.at[slot])
            pltpu.sync_copy(dst_hbm.at[pl.ds(base, CHM)], dst_idx.at[slot])
            pltpu.async_copy(mem_hbm.at[dst_idx.at[slot]], rows_d.at[slot],
                             sem_d[slot])
            pltpu.async_copy(mem_hbm.at[src_idx.at[slot]], rows_s.at[slot],
                             sem_s[slot])

    def drain(i, slot):
        j = i * NS + s

        @pl.when(j < cpc)
        def _():
            pltpu.make_async_copy(mem_hbm.at[dst_idx.at[slot]],
                                  rows_d.at[slot], sem_d[slot]).wait()
            pltpu.sync_copy(rows_d.at[slot], acc_m.at[src_idx.at[slot]],
                            add=True)
            pltpu.make_async_copy(mem_hbm.at[src_idx.at[slot]],
                                  rows_s.at[slot], sem_s[slot]).wait()
            pltpu.sync_copy(rows_s.at[slot], acc_m.at[dst_idx.at[slot]],
                            add=True)

    issue(0, 0)

    def body(i2, carry):
        for b2 in (0, 1):
            i = i2 * 2 + b2
            issue(i + 1, 1 - b2)
            drain(i, b2)
        return carry

    lax.fori_loop(0, (iters + 1) // 2, body, 0)
    plsc.subcore_barrier()
    pltpu.sync_copy(acc_m.at[pl.ds(r0, rows_per)],
                    accm_out.at[c, pl.ds(r0, rows_per)])


def _sc_scatter_pay_body(src_hbm, dst_hbm, pay_hbm, zm_hbm, accs_out,
                         src_idx, dst_idx, pay_buf, sem0, sem1, acc_s):
    n_pad = acc_s.shape[0]
    e = src_hbm.shape[0]
    c = lax.axis_index("c")
    s = lax.axis_index("s")
    rows_per = n_pad // NS
    r0 = s * rows_per
    pltpu.sync_copy(zm_hbm.at[pl.ds(r0, rows_per)], acc_s.at[pl.ds(r0, rows_per)])
    plsc.subcore_barrier()
    cpc = (e // CH) // NC
    iters = (cpc + NS - 1) // NS
    sems = (sem0, sem1)

    def issue(i, slot):
        j = i * NS + s

        @pl.when(j < cpc)
        def _():
            base = (c * cpc + j) * CH
            pltpu.sync_copy(src_hbm.at[pl.ds(base, CH)], src_idx.at[slot])
            pltpu.sync_copy(dst_hbm.at[pl.ds(base, CH)], dst_idx.at[slot])
            pltpu.async_copy(pay_hbm.at[pl.ds(base, CH)], pay_buf.at[slot],
                             sems[slot])

    def drain(i, slot):
        j = i * NS + s

        @pl.when(j < cpc)
        def _():
            base = (c * cpc + j) * CH
            pltpu.make_async_copy(pay_hbm.at[pl.ds(base, CH)],
                                  pay_buf.at[slot], sems[slot]).wait()
            pltpu.sync_copy(pay_buf.at[slot], acc_s.at[src_idx.at[slot]],
                            add=True)
            pltpu.sync_copy(pay_buf.at[slot], acc_s.at[dst_idx.at[slot]],
                            add=True)

    issue(0, 0)

    def body(i2, carry):
        for b2 in (0, 1):
            i = i2 * 2 + b2
            issue(i + 1, 1 - b2)
            drain(i, b2)
        return carry

    lax.fori_loop(0, (iters + 1) // 2, body, 0)
    plsc.subcore_barrier()
    pltpu.sync_copy(acc_s.at[pl.ds(r0, rows_per)],
                    accs_out.at[c, pl.ds(r0, rows_per)])


def _sc_scatter_mem(src, dst, memory, n, d):
    n_pad = ((n + 8 * NS - 1) // (8 * NS)) * (8 * NS)
    mesh = plsc.VectorSubcoreMesh(core_axis_name="c", subcore_axis_name="s")
    zm = jnp.zeros((n_pad, d), F32)
    fn = functools.partial(
        pl.kernel,
        out_type=jax.ShapeDtypeStruct((NC, n_pad, d), F32),
        mesh=mesh,
        scratch_types=[
            pltpu.VMEM((2, CHM), jnp.int32),
            pltpu.VMEM((2, CHM), jnp.int32),
            pltpu.VMEM((2, CHM, d), F32),
            pltpu.VMEM((2, CHM, d), F32),
            pltpu.SemaphoreType.DMA,
            pltpu.SemaphoreType.DMA,
            pltpu.SemaphoreType.DMA,
            pltpu.SemaphoreType.DMA,
            pltpu.VMEM_SHARED((n_pad, d), F32),
        ],
    )(_sc_scatter_mem_body)
    return fn(src, dst, memory, zm)


def _sc_scatter_pay(src, dst, payload, n, d):
    n_pad = ((n + 8 * NS - 1) // (8 * NS)) * (8 * NS)
    mesh = plsc.VectorSubcoreMesh(core_axis_name="c", subcore_axis_name="s")
    zm = jnp.zeros((n_pad, d), F32)
    fn = functools.partial(
        pl.kernel,
        out_type=jax.ShapeDtypeStruct((NC, n_pad, d), F32),
        mesh=mesh,
        scratch_types=[
            pltpu.VMEM((2, CH), jnp.int32),
            pltpu.VMEM((2, CH), jnp.int32),
            pltpu.VMEM((2, CH, d), F32),
            pltpu.SemaphoreType.DMA,
            pltpu.SemaphoreType.DMA,
            pltpu.VMEM_SHARED((n_pad, d), F32),
        ],
    )(_sc_scatter_pay_body)
    return fn(src, dst, payload, zm)


def _sc_gather_body(h_hbm, idx_hbm, out_hbm, idxb, rows, sem0, sem1):
    c = lax.axis_index("c")
    s = lax.axis_index("s")
    w = s * NC + c
    n_chunks = idx_hbm.shape[0] // CH
    iters = n_chunks // (NC * NS)
    sems = (sem0, sem1)

    def issue(i, slot):
        @pl.when(i < iters)
        def _():
            base = (i * NC * NS + w) * CH
            pltpu.sync_copy(idx_hbm.at[pl.ds(base, CH)], idxb.at[slot])
            pltpu.async_copy(h_hbm.at[idxb.at[slot]], rows.at[slot],
                             sems[slot])

    def drain(i, slot):
        @pl.when(i < iters)
        def _():
            base = (i * NC * NS + w) * CH
            pltpu.make_async_copy(h_hbm.at[idxb.at[slot]], rows.at[slot],
                                  sems[slot]).wait()
            pltpu.sync_copy(rows.at[slot], out_hbm.at[pl.ds(base, CH)])

    issue(0, 0)

    def body(i2, carry):
        for b2 in (0, 1):
            i = i2 * 2 + b2
            issue(i + 1, 1 - b2)
            drain(i, b2)
        return carry

    lax.fori_loop(0, (iters + 1) // 2, body, 0)


def _sc_gather(h_all, idx_all):
    n, d = h_all.shape
    m = idx_all.shape[0]
    mesh = plsc.VectorSubcoreMesh(core_axis_name="c", subcore_axis_name="s")
    fn = functools.partial(
        pl.kernel,
        out_type=jax.ShapeDtypeStruct((m, d), F32),
        mesh=mesh,
        scratch_types=[
            pltpu.VMEM((2, CH), jnp.int32),
            pltpu.VMEM((2, CH, d), F32),
            pltpu.SemaphoreType.DMA,
            pltpu.SemaphoreType.DMA,
        ],
    )(_sc_gather_body)
    return fn(h_all, idx_all)


def _payload_body(ef_ref, et_ref, w_ref, b_ref, out_ref):
    r = ef_ref.shape[0]
    te = _fast_cos(et_ref[...] * w_ref[...] + b_ref[...])
    out_ref[...] = jnp.concatenate(
        [ef_ref[...], te, jnp.ones((r, 1), F32), jnp.zeros((r, 95), F32)],
        axis=1)


def _tc_payload(event_feat, event_t16, time_w, time_b):
    e, de = event_feat.shape
    blk = 8000
    grid = e // blk
    return pl.pallas_call(
        _payload_body,
        grid=(grid,),
        in_specs=[
            pl.BlockSpec((blk, de), lambda i: (i, 0)),
            pl.BlockSpec((blk, 16), lambda i: (i, 0)),
            pl.BlockSpec((1, 16), lambda i: (0, 0)),
            pl.BlockSpec((1, 16), lambda i: (0, 0)),
        ],
        out_specs=pl.BlockSpec((blk, 128), lambda i: (i, 0)),
        out_shape=jax.ShapeDtypeStruct((e, 128), F32),
    )(event_feat, event_t16, time_w.reshape(1, 16),
      time_b.reshape(1, 16))


def _gru_body(accm_ref, accs_ref, mem_ref, nf_ref, wih_ref, bih_ref, whh_ref,
              bhh_ref, out_ref):
    am = accm_ref[0] + accm_ref[1]
    asml = accs_ref[0] + accs_ref[1]
    cnt = asml[:, 32:33]
    inv = 1.0 / jnp.maximum(cnt, 1.0)
    m = mem_ref[...]
    mm = jnp.concatenate([m, am * inv, asml[:, :32] * inv], axis=1)
    gi = jnp.dot(mm, wih_ref[...], preferred_element_type=F32) + bih_ref[...]
    gh = jnp.dot(m, whh_ref[...], preferred_element_type=F32) + bhh_ref[...]
    d = m.shape[1]
    r = jax.nn.sigmoid(gi[:, :d] + gh[:, :d])
    z = jax.nn.sigmoid(gi[:, d:2 * d] + gh[:, d:2 * d])
    nn = jnp.tanh(gi[:, 2 * d:] + r * gh[:, 2 * d:])
    new_mem = (1.0 - z) * nn + z * m
    out_ref[...] = jnp.where(cnt > 0, new_mem, m) + nf_ref[...]


def _tc_gru(accm, accs, memory, node_feat, w_ih, b_ih, w_hh, b_hh):
    n, d = memory.shape
    blk = 2000
    grid = n // blk
    return pl.pallas_call(
        _gru_body,
        grid=(grid,),
        in_specs=[
            pl.BlockSpec((NC, blk, d), lambda i: (0, i, 0)),
            pl.BlockSpec((NC, blk, d), lambda i: (0, i, 0)),
            pl.BlockSpec((blk, d), lambda i: (i, 0)),
            pl.BlockSpec((blk, d), lambda i: (i, 0)),
            pl.BlockSpec(w_ih.shape, lambda i: (0, 0)),
            pl.BlockSpec((1, 3 * d), lambda i: (0, 0)),
            pl.BlockSpec(w_hh.shape, lambda i: (0, 0)),
            pl.BlockSpec((1, 3 * d), lambda i: (0, 0)),
        ],
        out_specs=pl.BlockSpec((blk, d), lambda i: (i, 0)),
        out_shape=jax.ShapeDtypeStruct((n, d), F32),
    )(accm, accs, memory, node_feat, w_ih, b_ih.reshape(1, -1), w_hh,
      b_hh.reshape(1, -1))


def _attn_body(hs_ref, hn_ref, nef_ref, t_ref, nbt_ref, tw_ref, tb_ref,
               wq_ref, wk_ref, wv_ref, wo1_ref, wo2_ref, out_ref):
    r, d = hs_ref.shape
    k = nbt_ref.shape[1]
    hs = hs_ref[...]
    tw = tw_ref[...].reshape(1, 1, 16)
    tb = tb_ref[...].reshape(1, 1, 16)
    teq = _fast_cos(tb_ref[...])
    wq = wq_ref[...]
    q = (jnp.dot(hs, wq[:d], preferred_element_type=F32)
         + jnp.dot(teq, wq[d:], preferred_element_type=F32))
    dt = t_ref[...] - nbt_ref[...]
    ten = _fast_cos(dt[..., None] * tw + tb).reshape(r * k, 16)
    hn = hn_ref[...]
    nef = nef_ref[...]
    wk, wv = wk_ref[...], wv_ref[...]
    kk = (jnp.dot(hn, wk[:d], preferred_element_type=F32)
          + jnp.dot(nef, wk[d:d + 16], preferred_element_type=F32)
          + jnp.dot(ten, wk[d + 16:], preferred_element_type=F32))
    vv = (jnp.dot(hn, wv[:d], preferred_element_type=F32)
          + jnp.dot(nef, wv[d:d + 16], preferred_element_type=F32)
          + jnp.dot(ten, wv[d + 16:], preferred_element_type=F32))
    logits = jnp.sum(q[:, None, :] * kk.reshape(r, k, d), axis=-1)
    logits = logits * (1.0 / jnp.sqrt(jnp.float32(d)))
    mx = jnp.max(logits, axis=-1, keepdims=True)
    ex = jnp.exp(logits - mx)
    attn = ex / jnp.sum(ex, axis=-1, keepdims=True)
    out = jnp.sum(attn[..., None] * vv.reshape(r, k, d), axis=1)
    wo1 = wo1_ref[...]
    hcat = (jnp.dot(out, wo1[:d], preferred_element_type=F32)
            + jnp.dot(hs, wo1[d:], preferred_element_type=F32))
    emb = jnp.dot(jax.nn.relu(hcat), wo2_ref[...],
                  preferred_element_type=F32)
    out_ref[...] = emb


def _tc_attn(h_src, h_nbr_flat, nbr_ef_flat, t, nbr_t, time_w, time_b,
             w_q, w_k, w_v, w_o1, w_o2):
    b, d = h_src.shape
    k = nbr_t.shape[1]
    blk = 512
    grid = b // blk
    return pl.pallas_call(
        _attn_body,
        grid=(grid,),
        in_specs=[
            pl.BlockSpec((blk, d), lambda i: (i, 0)),
            pl.BlockSpec((blk * k, d), lambda i: (i, 0)),
            pl.BlockSpec((blk * k, 16), lambda i: (i, 0)),
            pl.BlockSpec((blk, k), lambda i: (i, 0)),
            pl.BlockSpec((blk, k), lambda i: (i, 0)),
            pl.BlockSpec((1, 16), lambda i: (0, 0)),
            pl.BlockSpec((1, 16), lambda i: (0, 0)),
            pl.BlockSpec(w_q.shape, lambda i: (0, 0)),
            pl.BlockSpec(w_k.shape, lambda i: (0, 0)),
            pl.BlockSpec(w_v.shape, lambda i: (0, 0)),
            pl.BlockSpec(w_o1.shape, lambda i: (0, 0)),
            pl.BlockSpec(w_o2.shape, lambda i: (0, 0)),
        ],
        out_specs=pl.BlockSpec((blk, d), lambda i: (i, 0)),
        out_shape=jax.ShapeDtypeStruct((b, d), F32),
    )(h_src, h_nbr_flat, nbr_ef_flat, jnp.broadcast_to(t[:, None], (b, k)),
      nbr_t, time_w.reshape(1, 16), time_b.reshape(1, 16), w_q, w_k, w_v,
      w_o1, w_o2)


def kernel(idx, t, src, dst, event_t, event_feat, nbr_idx, nbr_t,
           nbr_edge_feat, memory, node_feat, time_w, time_b, W_ih, b_ih,
           W_hh, b_hh, W_q, W_k, W_v, W_o1, W_o2):
    b, k = nbr_idx.shape
    n, d = memory.shape
    e = src.shape[0]
    accm = _sc_scatter_mem(src, dst, memory, n, d)
    et16 = jnp.broadcast_to(event_t[:, None], (e, 16))
    payload = _tc_payload(event_feat, et16, time_w, time_b)
    accs = _sc_scatter_pay(src, dst, payload, n, d)
    h_all = _tc_gru(accm, accs, memory, node_feat, W_ih, b_ih, W_hh, b_hh)
    idx_all = jnp.concatenate([idx, nbr_idx.reshape(-1)])
    rows = _sc_gather(h_all, idx_all)
    h_src = rows[:b]
    h_nbr_flat = rows[b:]
    emb = _tc_attn(h_src, h_nbr_flat, nbr_edge_feat.reshape(b * k, 16),
                   t, nbr_t, time_w, time_b, W_q, W_k, W_v, W_o1, W_o2)
    return emb


# speedup vs baseline: 5.6476x; 1.0440x over previous
import functools

import jax
import jax.numpy as jnp
from jax import lax
from jax.experimental import pallas as pl
from jax.experimental.pallas import tpu as pltpu
from jax.experimental.pallas import tpu_sc as plsc

NC, NS = 2, 16
CH = 128
CHM = 64

F32 = jnp.float32

_COS_C = (0.999999999708885, -19.739208718081116, 64.9393881176947,
          -85.45664332355348, 60.24201925362776, -26.404267783730656,
          7.799566579651471, -1.4530463585373052)


def _fast_cos(x):
    u = x * 0.15915494309189535
    u = u - jnp.floor(u + 0.5)
    v = u * u
    p = jnp.float32(_COS_C[7])
    for cc in _COS_C[6::-1]:
        p = p * v + jnp.float32(cc)
    return p


def _sc_scatter_mem_body(src_hbm, dst_hbm, mem_hbm, zm_hbm, accm_out,
                         src_idx, dst_idx, rows_s, rows_d,
                         sem_s0, sem_s1, sem_d0, sem_d1, acc_m):
    n_pad = acc_m.shape[0]
    e = src_hbm.shape[0]
    c = lax.axis_index("c")
    s = lax.axis_index("s")
    rows_per = n_pad // NS
    r0 = s * rows_per
    pltpu.sync_copy(zm_hbm.at[pl.ds(r0, rows_per)], acc_m.at[pl.ds(r0, rows_per)])
    plsc.subcore_barrier()
    cpc = (e // CHM) // NC
    iters = (cpc + NS - 1) // NS
    sem_s = (sem_s0, sem_s1)
    sem_d = (sem_d0, sem_d1)

    def issue(i, slot):
        j = i * NS + s

        @pl.when(j < cpc)
        def _():
            base = (c * cpc + j) * CHM
            pltpu.sync_copy(src_hbm.at[pl.ds(base, CHM)], src_idx.at[slot])
            pltpu.sync_copy(dst_hbm.at[pl.ds(base, CHM)], dst_idx.at[slot])
            pltpu.async_copy(mem_hbm.at[dst_idx.at[slot]], rows_d.at[slot],
                             sem_d[slot])
            pltpu.async_copy(mem_hbm.at[src_idx.at[slot]], rows_s.at[slot],
                             sem_s[slot])

    def drain(i, slot):
        j = i * NS + s

        @pl.when(j < cpc)
        def _():
            pltpu.make_async_copy(mem_hbm.at[dst_idx.at[slot]],
                                  rows_d.at[slot], sem_d[slot]).wait()
            pltpu.sync_copy(rows_d.at[slot], acc_m.at[src_idx.at[slot]],
                            add=True)
            pltpu.make_async_copy(mem_hbm.at[src_idx.at[slot]],
                                  rows_s.at[slot], sem_s[slot]).wait()
            pltpu.sync_copy(rows_s.at[slot], acc_m.at[dst_idx.at[slot]],
                            add=True)

    issue(0, 0)

    def body(i2, carry):
        for b2 in (0, 1):
            i = i2 * 2 + b2
            issue(i + 1, 1 - b2)
            drain(i, b2)
        return carry

    lax.fori_loop(0, (iters + 1) // 2, body, 0)
    plsc.subcore_barrier()
    pltpu.sync_copy(acc_m.at[pl.ds(r0, rows_per)],
                    accm_out.at[c, pl.ds(r0, rows_per)])


def _sc_scatter_pay_body(src_hbm, dst_hbm, pay_hbm, zm_hbm, accs_out,
                         src_idx, dst_idx, pay_buf, sem0, sem1, acc_s):
    n_pad = acc_s.shape[0]
    e = src_hbm.shape[0]
    c = lax.axis_index("c")
    s = lax.axis_index("s")
    rows_per = n_pad // NS
    r0 = s * rows_per
    pltpu.sync_copy(zm_hbm.at[pl.ds(r0, rows_per)], acc_s.at[pl.ds(r0, rows_per)])
    plsc.subcore_barrier()
    cpc = (e // CH) // NC
    iters = (cpc + NS - 1) // NS
    sems = (sem0, sem1)

    def issue(i, slot):
        j = i * NS + s

        @pl.when(j < cpc)
        def _():
            base = (c * cpc + j) * CH
            pltpu.sync_copy(src_hbm.at[pl.ds(base, CH)], src_idx.at[slot])
            pltpu.sync_copy(dst_hbm.at[pl.ds(base, CH)], dst_idx.at[slot])
            pltpu.async_copy(pay_hbm.at[pl.ds(base, CH)], pay_buf.at[slot],
                             sems[slot])

    def drain(i, slot):
        j = i * NS + s

        @pl.when(j < cpc)
        def _():
            base = (c * cpc + j) * CH
            pltpu.make_async_copy(pay_hbm.at[pl.ds(base, CH)],
                                  pay_buf.at[slot], sems[slot]).wait()
            pltpu.sync_copy(pay_buf.at[slot], acc_s.at[src_idx.at[slot]],
                            add=True)
            pltpu.sync_copy(pay_buf.at[slot], acc_s.at[dst_idx.at[slot]],
                            add=True)

    issue(0, 0)

    def body(i2, carry):
        for b2 in (0, 1):
            i = i2 * 2 + b2
            issue(i + 1, 1 - b2)
            drain(i, b2)
        return carry

    lax.fori_loop(0, (iters + 1) // 2, body, 0)
    plsc.subcore_barrier()
    pltpu.sync_copy(acc_s.at[pl.ds(r0, rows_per)],
                    accs_out.at[c, pl.ds(r0, rows_per)])


def _sc_scatter_mem(src, dst, memory, n, d):
    n_pad = ((n + 8 * NS - 1) // (8 * NS)) * (8 * NS)
    mesh = plsc.VectorSubcoreMesh(core_axis_name="c", subcore_axis_name="s")
    zm = jnp.zeros((n_pad, d), F32)
    fn = functools.partial(
        pl.kernel,
        out_type=jax.ShapeDtypeStruct((NC, n_pad, d), F32),
        mesh=mesh,
        scratch_types=[
            pltpu.VMEM((2, CHM), jnp.int32),
            pltpu.VMEM((2, CHM), jnp.int32),
            pltpu.VMEM((2, CHM, d), F32),
            pltpu.VMEM((2, CHM, d), F32),
            pltpu.SemaphoreType.DMA,
            pltpu.SemaphoreType.DMA,
            pltpu.SemaphoreType.DMA,
            pltpu.SemaphoreType.DMA,
            pltpu.VMEM_SHARED((n_pad, d), F32),
        ],
    )(_sc_scatter_mem_body)
    return fn(src, dst, memory, zm)


def _sc_scatter_pay(src, dst, payload, n, d):
    n_pad = ((n + 8 * NS - 1) // (8 * NS)) * (8 * NS)
    mesh = plsc.VectorSubcoreMesh(core_axis_name="c", subcore_axis_name="s")
    zm = jnp.zeros((n_pad, d), F32)
    fn = functools.partial(
        pl.kernel,
        out_type=jax.ShapeDtypeStruct((NC, n_pad, d), F32),
        mesh=mesh,
        scratch_types=[
            pltpu.VMEM((2, CH), jnp.int32),
            pltpu.VMEM((2, CH), jnp.int32),
            pltpu.VMEM((2, CH, d), F32),
            pltpu.SemaphoreType.DMA,
            pltpu.SemaphoreType.DMA,
            pltpu.VMEM_SHARED((n_pad, d), F32),
        ],
    )(_sc_scatter_pay_body)
    return fn(src, dst, payload, zm)


def _sc_gather_body(h_hbm, qidx_hbm, nidx_hbm, outq_hbm, outn_hbm,
                    idxb, rows, sem0, sem1):
    c = lax.axis_index("c")
    s = lax.axis_index("s")
    w = s * NC + c
    nw = NC * NS
    iters_n = (nidx_hbm.shape[0] // CH) // nw
    sems = (sem0, sem1)

    def issue(i, slot):
        @pl.when(i <= iters_n)
        def _():
            qbase = w * CH
            nbase = ((i - 1) * nw + w) * CH
            @pl.when(i == 0)
            def _():
                pltpu.sync_copy(qidx_hbm.at[pl.ds(qbase, CH)], idxb.at[slot])
            @pl.when(i > 0)
            def _():
                pltpu.sync_copy(nidx_hbm.at[pl.ds(nbase, CH)], idxb.at[slot])
            pltpu.async_copy(h_hbm.at[idxb.at[slot]], rows.at[slot],
                             sems[slot])

    def drain(i, slot):
        @pl.when(i <= iters_n)
        def _():
            pltpu.make_async_copy(h_hbm.at[idxb.at[slot]], rows.at[slot],
                                  sems[slot]).wait()
            @pl.when(i == 0)
            def _():
                pltpu.sync_copy(rows.at[slot], outq_hbm.at[pl.ds(w * CH, CH)])
            @pl.when(i > 0)
            def _():
                nbase = ((i - 1) * nw + w) * CH
                pltpu.sync_copy(rows.at[slot], outn_hbm.at[pl.ds(nbase, CH)])

    issue(0, 0)

    def body(i2, carry):
        for b2 in (0, 1):
            i = i2 * 2 + b2
            issue(i + 1, 1 - b2)
            drain(i, b2)
        return carry

    lax.fori_loop(0, (iters_n + 2) // 2, body, 0)


def _sc_gather(h_all, qidx, nidx):
    n, d = h_all.shape
    mesh = plsc.VectorSubcoreMesh(core_axis_name="c", subcore_axis_name="s")
    fn = functools.partial(
        pl.kernel,
        out_type=[jax.ShapeDtypeStruct((qidx.shape[0], d), F32),
                  jax.ShapeDtypeStruct((nidx.shape[0], d), F32)],
        mesh=mesh,
        scratch_types=[
            pltpu.VMEM((2, CH), jnp.int32),
            pltpu.VMEM((2, CH, d), F32),
            pltpu.SemaphoreType.DMA,
            pltpu.SemaphoreType.DMA,
        ],
    )(_sc_gather_body)
    return fn(h_all, qidx, nidx)


def _payload_body(ef_ref, et_ref, w_ref, b_ref, out_ref):
    r = ef_ref.shape[0]
    te = _fast_cos(et_ref[...] * w_ref[...] + b_ref[...])
    out_ref[...] = jnp.concatenate(
        [ef_ref[...], te, jnp.ones((r, 1), F32), jnp.zeros((r, 95), F32)],
        axis=1)


def _tc_payload(event_feat, event_t16, time_w, time_b):
    e, de = event_feat.shape
    blk = 8000
    grid = e // blk
    return pl.pallas_call(
        _payload_body,
        grid=(grid,),
        in_specs=[
            pl.BlockSpec((blk, de), lambda i: (i, 0)),
            pl.BlockSpec((blk, 16), lambda i: (i, 0)),
            pl.BlockSpec((1, 16), lambda i: (0, 0)),
            pl.BlockSpec((1, 16), lambda i: (0, 0)),
        ],
        out_specs=pl.BlockSpec((blk, 128), lambda i: (i, 0)),
        out_shape=jax.ShapeDtypeStruct((e, 128), F32),
    )(event_feat, event_t16, time_w.reshape(1, 16),
      time_b.reshape(1, 16))


def _gru_body(accm_ref, accs_ref, mem_ref, nf_ref, wih_ref, bih_ref, whh_ref,
              bhh_ref, out_ref):
    am = accm_ref[0] + accm_ref[1]
    asml = accs_ref[0] + accs_ref[1]
    cnt = asml[:, 32:33]
    inv = 1.0 / jnp.maximum(cnt, 1.0)
    m = mem_ref[...]
    mm = jnp.concatenate([m, am * inv, asml[:, :32] * inv], axis=1)
    gi = jnp.dot(mm, wih_ref[...], preferred_element_type=F32) + bih_ref[...]
    gh = jnp.dot(m, whh_ref[...], preferred_element_type=F32) + bhh_ref[...]
    d = m.shape[1]
    r = jax.nn.sigmoid(gi[:, :d] + gh[:, :d])
    z = jax.nn.sigmoid(gi[:, d:2 * d] + gh[:, d:2 * d])
    nn = jnp.tanh(gi[:, 2 * d:] + r * gh[:, 2 * d:])
    new_mem = (1.0 - z) * nn + z * m
    out_ref[...] = jnp.where(cnt > 0, new_mem, m) + nf_ref[...]


def _tc_gru(accm, accs, memory, node_feat, w_ih, b_ih, w_hh, b_hh):
    n, d = memory.shape
    blk = 2000
    grid = n // blk
    return pl.pallas_call(
        _gru_body,
        grid=(grid,),
        in_specs=[
            pl.BlockSpec((NC, blk, d), lambda i: (0, i, 0)),
            pl.BlockSpec((NC, blk, d), lambda i: (0, i, 0)),
            pl.BlockSpec((blk, d), lambda i: (i, 0)),
            pl.BlockSpec((blk, d), lambda i: (i, 0)),
            pl.BlockSpec(w_ih.shape, lambda i: (0, 0)),
            pl.BlockSpec((1, 3 * d), lambda i: (0, 0)),
            pl.BlockSpec(w_hh.shape, lambda i: (0, 0)),
            pl.BlockSpec((1, 3 * d), lambda i: (0, 0)),
        ],
        out_specs=pl.BlockSpec((blk, d), lambda i: (i, 0)),
        out_shape=jax.ShapeDtypeStruct((n, d), F32),
    )(accm, accs, memory, node_feat, w_ih, b_ih.reshape(1, -1), w_hh,
      b_hh.reshape(1, -1))


def _attn_body(hs_ref, hn_ref, nef_ref, t_ref, nbt_ref, tw_ref, tb_ref,
               wq_ref, wk_ref, wv_ref, wo1_ref, wo2_ref, out_ref):
    r, d = hs_ref.shape
    k = nbt_ref.shape[1]
    hs = hs_ref[...]
    tw = tw_ref[...].reshape(1, 1, 16)
    tb = tb_ref[...].reshape(1, 1, 16)
    teq = _fast_cos(tb_ref[...])
    wq = wq_ref[...]
    q = (jnp.dot(hs, wq[:d], preferred_element_type=F32)
         + jnp.dot(teq, wq[d:], preferred_element_type=F32))
    dt = t_ref[...] - nbt_ref[...]
    ten = _fast_cos(dt[..., None] * tw + tb).reshape(r * k, 16)
    hn = hn_ref[...]
    nef = nef_ref[...].reshape(r * k, 16)
    wk, wv = wk_ref[...], wv_ref[...]
    kk = (jnp.dot(hn, wk[:d], preferred_element_type=F32)
          + jnp.dot(nef, wk[d:d + 16], preferred_element_type=F32)
          + jnp.dot(ten, wk[d + 16:], preferred_element_type=F32))
    vv = (jnp.dot(hn, wv[:d], preferred_element_type=F32)
          + jnp.dot(nef, wv[d:d + 16], preferred_element_type=F32)
          + jnp.dot(ten, wv[d + 16:], preferred_element_type=F32))
    logits = jnp.sum(q[:, None, :] * kk.reshape(r, k, d), axis=-1)
    logits = logits * (1.0 / jnp.sqrt(jnp.float32(d)))
    mx = jnp.max(logits, axis=-1, keepdims=True)
    ex = jnp.exp(logits - mx)
    attn = ex / jnp.sum(ex, axis=-1, keepdims=True)
    out = jnp.sum(attn[..., None] * vv.reshape(r, k, d), axis=1)
    wo1 = wo1_ref[...]
    hcat = (jnp.dot(out, wo1[:d], preferred_element_type=F32)
            + jnp.dot(hs, wo1[d:], preferred_element_type=F32))
    emb = jnp.dot(jax.nn.relu(hcat), wo2_ref[...],
                  preferred_element_type=F32)
    out_ref[...] = emb


def _tc_attn(h_src, h_nbr_flat, nbr_ef_flat, t, nbr_t, time_w, time_b,
             w_q, w_k, w_v, w_o1, w_o2):
    b, d = h_src.shape
    k = nbr_t.shape[1]
    blk = 512
    grid = b // blk
    return pl.pallas_call(
        _attn_body,
        grid=(grid,),
        in_specs=[
            pl.BlockSpec((blk, d), lambda i: (i, 0)),
            pl.BlockSpec((blk * k, d), lambda i: (i, 0)),
            pl.BlockSpec((blk, k, 16), lambda i: (i, 0, 0)),
            pl.BlockSpec((blk, k), lambda i: (i, 0)),
            pl.BlockSpec((blk, k), lambda i: (i, 0)),
            pl.BlockSpec((1, 16), lambda i: (0, 0)),
            pl.BlockSpec((1, 16), lambda i: (0, 0)),
            pl.BlockSpec(w_q.shape, lambda i: (0, 0)),
            pl.BlockSpec(w_k.shape, lambda i: (0, 0)),
            pl.BlockSpec(w_v.shape, lambda i: (0, 0)),
            pl.BlockSpec(w_o1.shape, lambda i: (0, 0)),
            pl.BlockSpec(w_o2.shape, lambda i: (0, 0)),
        ],
        out_specs=pl.BlockSpec((blk, d), lambda i: (i, 0)),
        out_shape=jax.ShapeDtypeStruct((b, d), F32),
    )(h_src, h_nbr_flat, nbr_ef_flat, jnp.broadcast_to(t[:, None], (b, k)),
      nbr_t, time_w.reshape(1, 16), time_b.reshape(1, 16), w_q, w_k, w_v,
      w_o1, w_o2)


def kernel(idx, t, src, dst, event_t, event_feat, nbr_idx, nbr_t,
           nbr_edge_feat, memory, node_feat, time_w, time_b, W_ih, b_ih,
           W_hh, b_hh, W_q, W_k, W_v, W_o1, W_o2):
    b, k = nbr_idx.shape
    n, d = memory.shape
    e = src.shape[0]
    accm = _sc_scatter_mem(src, dst, memory, n, d)
    et16 = jnp.broadcast_to(event_t[:, None], (e, 16))
    payload = _tc_payload(event_feat, et16, time_w, time_b)
    accs = _sc_scatter_pay(src, dst, payload, n, d)
    h_all = _tc_gru(accm, accs, memory, node_feat, W_ih, b_ih, W_hh, b_hh)
    h_src, h_nbr_flat = _sc_gather(h_all, idx, nbr_idx.reshape(-1))
    emb = _tc_attn(h_src, h_nbr_flat, nbr_edge_feat,
                   t, nbr_t, time_w, time_b, W_q, W_k, W_v, W_o1, W_o2)
    return emb


# speedup vs baseline: 6.3441x; 1.1233x over previous
import functools

import jax
import jax.numpy as jnp
from jax import lax
from jax.experimental import pallas as pl
from jax.experimental.pallas import tpu as pltpu
from jax.experimental.pallas import tpu_sc as plsc

NC, NS = 2, 16
CH = 128
CHM = 64
CHP = 64

F32 = jnp.float32

_COS_C = (0.999999999708885, -19.739208718081116, 64.9393881176947,
          -85.45664332355348, 60.24201925362776, -26.404267783730656,
          7.799566579651471, -1.4530463585373052)


def _fast_cos(x):
    u = x * 0.15915494309189535
    u = u - jnp.floor(u + 0.5)
    v = u * u
    p = jnp.float32(_COS_C[7])
    for cc in _COS_C[6::-1]:
        p = p * v + jnp.float32(cc)
    return p


def _sc_scatter_mem_body(src_hbm, dst_hbm, mem_hbm, zm_hbm, accm_out,
                         src_idx, dst_idx, rows_s, rows_d,
                         sem_s0, sem_s1, sem_d0, sem_d1, acc_m):
    n_pad = acc_m.shape[0]
    e = src_hbm.shape[0]
    c = lax.axis_index("c")
    s = lax.axis_index("s")
    rows_per = n_pad // NS
    r0 = s * rows_per
    pltpu.sync_copy(zm_hbm.at[pl.ds(r0, rows_per)], acc_m.at[pl.ds(r0, rows_per)])
    plsc.subcore_barrier()
    cpc = (e // CHM) // NC
    iters = (cpc + NS - 1) // NS
    sem_s = (sem_s0, sem_s1)
    sem_d = (sem_d0, sem_d1)

    def issue(i, slot):
        j = i * NS + s

        @pl.when(j < cpc)
        def _():
            base = (c * cpc + j) * CHM
            pltpu.sync_copy(src_hbm.at[pl.ds(base, CHM)], src_idx.at[slot])
            pltpu.sync_copy(dst_hbm.at[pl.ds(base, CHM)], dst_idx.at[slot])
            pltpu.async_copy(mem_hbm.at[dst_idx.at[slot]], rows_d.at[slot],
                             sem_d[slot])
            pltpu.async_copy(mem_hbm.at[src_idx.at[slot]], rows_s.at[slot],
                             sem_s[slot])

    def drain(i, slot):
        j = i * NS + s

        @pl.when(j < cpc)
        def _():
            pltpu.make_async_copy(mem_hbm.at[dst_idx.at[slot]],
                                  rows_d.at[slot], sem_d[slot]).wait()
            pltpu.sync_copy(rows_d.at[slot], acc_m.at[src_idx.at[slot]],
                            add=True)
            pltpu.make_async_copy(mem_hbm.at[src_idx.at[slot]],
                                  rows_s.at[slot], sem_s[slot]).wait()
            pltpu.sync_copy(rows_s.at[slot], acc_m.at[dst_idx.at[slot]],
                            add=True)

    issue(0, 0)

    def body(i2, carry):
        for b2 in (0, 1):
            i = i2 * 2 + b2
            issue(i + 1, 1 - b2)
            drain(i, b2)
        return carry

    lax.fori_loop(0, (iters + 1) // 2, body, 0)
    plsc.subcore_barrier()
    pltpu.sync_copy(acc_m.at[pl.ds(r0, rows_per)],
                    accm_out.at[c, pl.ds(r0, rows_per)])


def _fast_cos_sc(x):
    u = x * 0.15915494309189535
    y = u + 0.5
    tr = y.astype(jnp.int32).astype(F32)
    fl = jnp.where(tr > y, tr - 1.0, tr)
    u = u - fl
    v = u * u
    p = jnp.full_like(v, _COS_C[7])
    for cc in _COS_C[6::-1]:
        p = p * v + jnp.float32(cc)
    return p


def _sc_scatter_pay_body(src_hbm, dst_hbm, et_hbm, ef_hbm, tw_hbm, tb_hbm,
                         zm_hbm, accs_out,
                         src_idx, dst_idx, etb, efb, pay_buf, twb, tbb,
                         sem0, sem1, acc_s):
    n_pad = acc_s.shape[0]
    e = src_hbm.shape[0]
    c = lax.axis_index("c")
    s = lax.axis_index("s")
    rows_per = n_pad // NS
    r0 = s * rows_per
    pltpu.sync_copy(zm_hbm.at[pl.ds(r0, rows_per)], acc_s.at[pl.ds(r0, rows_per)])
    pltpu.sync_copy(tw_hbm, twb)
    pltpu.sync_copy(tb_hbm, tbb)
    one0 = jnp.where(lax.iota(jnp.int32, 16) == 0, 1.0, 0.0).astype(F32)
    zero16 = jnp.zeros((16,), F32)
    for slot in (0, 1):
        for i in range(CHP):
            pay_buf[slot, i, pl.ds(32, 16)] = one0
            for col in range(48, 128, 16):
                pay_buf[slot, i, pl.ds(col, 16)] = zero16
    plsc.subcore_barrier()
    cpc = (e // CHP) // NC
    iters = (cpc + NS - 1) // NS
    sems = (sem0, sem1)
    wv = twb[...]
    bv = tbb[...]

    def issue(i, slot):
        j = i * NS + s

        @pl.when(j < cpc)
        def _():
            base = (c * cpc + j) * CHP
            pltpu.sync_copy(src_hbm.at[pl.ds(base, CHP)], src_idx.at[slot])
            pltpu.sync_copy(dst_hbm.at[pl.ds(base, CHP)], dst_idx.at[slot])
            pltpu.sync_copy(et_hbm.at[pl.ds(base, CHP)], etb.at[slot])
            efrow = pl.multiple_of(base * 16 // 128, 8)
            pltpu.async_copy(ef_hbm.at[pl.ds(efrow, CHP * 16 // 128)],
                             efb.at[slot], sems[slot])

    def drain(i, slot):
        j = i * NS + s

        @pl.when(j < cpc)
        def _():
            pltpu.make_async_copy(
                ef_hbm.at[pl.ds(0, CHP * 16 // 128)], efb.at[slot],
                sems[slot]).wait()
            for g in range(CHP // 16):
                tv = etb[slot, pl.ds(g * 16, 16)]
                for l in range(16):
                    i_ev = g * 16 + l
                    te = _fast_cos_sc(tv[l] * wv + bv)
                    pay_buf[slot, i_ev, pl.ds(16, 16)] = te
                    pay_buf[slot, i_ev, pl.ds(0, 16)] = (
                        efb[slot, i_ev // 8, pl.ds(16 * (i_ev % 8), 16)])
            pltpu.sync_copy(pay_buf.at[slot], acc_s.at[src_idx.at[slot]],
                            add=True)
            pltpu.sync_copy(pay_buf.at[slot], acc_s.at[dst_idx.at[slot]],
                            add=True)

    issue(0, 0)

    def body(i2, carry):
        for b2 in (0, 1):
            i = i2 * 2 + b2
            issue(i + 1, 1 - b2)
            drain(i, b2)
        return carry

    lax.fori_loop(0, (iters + 1) // 2, body, 0)
    plsc.subcore_barrier()
    pltpu.sync_copy(acc_s.at[pl.ds(r0, rows_per)],
                    accs_out.at[c, pl.ds(r0, rows_per)])


def _sc_scatter_mem(src, dst, memory, n, d):
    n_pad = ((n + 8 * NS - 1) // (8 * NS)) * (8 * NS)
    mesh = plsc.VectorSubcoreMesh(core_axis_name="c", subcore_axis_name="s")
    zm = jnp.zeros((n_pad, d), F32)
    fn = functools.partial(
        pl.kernel,
        out_type=jax.ShapeDtypeStruct((NC, n_pad, d), F32),
        mesh=mesh,
        scratch_types=[
            pltpu.VMEM((2, CHM), jnp.int32),
            pltpu.VMEM((2, CHM), jnp.int32),
            pltpu.VMEM((2, CHM, d), F32),
            pltpu.VMEM((2, CHM, d), F32),
            pltpu.SemaphoreType.DMA,
            pltpu.SemaphoreType.DMA,
            pltpu.SemaphoreType.DMA,
            pltpu.SemaphoreType.DMA,
            pltpu.VMEM_SHARED((n_pad, d), F32),
        ],
    )(_sc_scatter_mem_body)
    return fn(src, dst, memory, zm)


def _sc_scatter_pay(src, dst, event_t, event_feat, time_w, time_b, n, d):
    n_pad = ((n + 8 * NS - 1) // (8 * NS)) * (8 * NS)
    e = src.shape[0]
    mesh = plsc.VectorSubcoreMesh(core_axis_name="c", subcore_axis_name="s")
    zm = jnp.zeros((n_pad, d), F32)
    ef128 = event_feat.reshape(e * 16 // 128, 128)
    fn = functools.partial(
        pl.kernel,
        out_type=jax.ShapeDtypeStruct((NC, n_pad, d), F32),
        mesh=mesh,
        scratch_types=[
            pltpu.VMEM((2, CHP), jnp.int32),
            pltpu.VMEM((2, CHP), jnp.int32),
            pltpu.VMEM((2, CHP), F32),
            pltpu.VMEM((2, CHP * 16 // 128, 128), F32),
            pltpu.VMEM((2, CHP, d), F32),
            pltpu.VMEM((16,), F32),
            pltpu.VMEM((16,), F32),
            pltpu.SemaphoreType.DMA,
            pltpu.SemaphoreType.DMA,
            pltpu.VMEM_SHARED((n_pad, d), F32),
        ],
    )(_sc_scatter_pay_body)
    return fn(src, dst, event_t, ef128, time_w, time_b, zm)


def _sc_gather_body(h_hbm, qidx_hbm, nidx_hbm, outq_hbm, outn_hbm,
                    idxb, rows, sem0, sem1):
    c = lax.axis_index("c")
    s = lax.axis_index("s")
    w = s * NC + c
    nw = NC * NS
    iters_n = (nidx_hbm.shape[0] // CH) // nw
    sems = (sem0, sem1)

    def issue(i, slot):
        @pl.when(i <= iters_n)
        def _():
            qbase = w * CH
            nbase = ((i - 1) * nw + w) * CH
            @pl.when(i == 0)
            def _():
                pltpu.sync_copy(qidx_hbm.at[pl.ds(qbase, CH)], idxb.at[slot])
            @pl.when(i > 0)
            def _():
                pltpu.sync_copy(nidx_hbm.at[pl.ds(nbase, CH)], idxb.at[slot])
            pltpu.async_copy(h_hbm.at[idxb.at[slot]], rows.at[slot],
                             sems[slot])

    def drain(i, slot):
        @pl.when(i <= iters_n)
        def _():
            pltpu.make_async_copy(h_hbm.at[idxb.at[slot]], rows.at[slot],
                                  sems[slot]).wait()
            @pl.when(i == 0)
            def _():
                pltpu.sync_copy(rows.at[slot], outq_hbm.at[pl.ds(w * CH, CH)])
            @pl.when(i > 0)
            def _():
                nbase = ((i - 1) * nw + w) * CH
                pltpu.sync_copy(rows.at[slot], outn_hbm.at[pl.ds(nbase, CH)])

    issue(0, 0)

    def body(i2, carry):
        for b2 in (0, 1):
            i = i2 * 2 + b2
            issue(i + 1, 1 - b2)
            drain(i, b2)
        return carry

    lax.fori_loop(0, (iters_n + 2) // 2, body, 0)


def _sc_gather(h_all, qidx, nidx):
    n, d = h_all.shape
    mesh = plsc.VectorSubcoreMesh(core_axis_name="c", subcore_axis_name="s")
    fn = functools.partial(
        pl.kernel,
        out_type=[jax.ShapeDtypeStruct((qidx.shape[0], d), F32),
                  jax.ShapeDtypeStruct((nidx.shape[0], d), F32)],
        mesh=mesh,
        scratch_types=[
            pltpu.VMEM((2, CH), jnp.int32),
            pltpu.VMEM((2, CH, d), F32),
            pltpu.SemaphoreType.DMA,
            pltpu.SemaphoreType.DMA,
        ],
    )(_sc_gather_body)
    return fn(h_all, qidx, nidx)


def _gru_body(accm_ref, accs_ref, mem_ref, nf_ref, wih_ref, bih_ref, whh_ref,
              bhh_ref, out_ref):
    am = accm_ref[0] + accm_ref[1]
    asml = accs_ref[0] + accs_ref[1]
    cnt = asml[:, 32:33]
    inv = 1.0 / jnp.maximum(cnt, 1.0)
    m = mem_ref[...]
    mm = jnp.concatenate([m, am * inv, asml[:, :32] * inv], axis=1)
    gi = jnp.dot(mm, wih_ref[...], preferred_element_type=F32) + bih_ref[...]
    gh = jnp.dot(m, whh_ref[...], preferred_element_type=F32) + bhh_ref[...]
    d = m.shape[1]
    r = jax.nn.sigmoid(gi[:, :d] + gh[:, :d])
    z = jax.nn.sigmoid(gi[:, d:2 * d] + gh[:, d:2 * d])
    nn = jnp.tanh(gi[:, 2 * d:] + r * gh[:, 2 * d:])
    new_mem = (1.0 - z) * nn + z * m
    out_ref[...] = jnp.where(cnt > 0, new_mem, m) + nf_ref[...]


def _tc_gru(accm, accs, memory, node_feat, w_ih, b_ih, w_hh, b_hh):
    n, d = memory.shape
    blk = 2000
    grid = n // blk
    return pl.pallas_call(
        _gru_body,
        grid=(grid,),
        in_specs=[
            pl.BlockSpec((NC, blk, d), lambda i: (0, i, 0)),
            pl.BlockSpec((NC, blk, d), lambda i: (0, i, 0)),
            pl.BlockSpec((blk, d), lambda i: (i, 0)),
            pl.BlockSpec((blk, d), lambda i: (i, 0)),
            pl.BlockSpec(w_ih.shape, lambda i: (0, 0)),
            pl.BlockSpec((1, 3 * d), lambda i: (0, 0)),
            pl.BlockSpec(w_hh.shape, lambda i: (0, 0)),
            pl.BlockSpec((1, 3 * d), lambda i: (0, 0)),
        ],
        out_specs=pl.BlockSpec((blk, d), lambda i: (i, 0)),
        out_shape=jax.ShapeDtypeStruct((n, d), F32),
    )(accm, accs, memory, node_feat, w_ih, b_ih.reshape(1, -1), w_hh,
      b_hh.reshape(1, -1))


def _attn_body(hs_ref, hn_ref, nef_ref, t_ref, nbt_ref, tw_ref, tb_ref,
               wq_ref, wk_ref, wv_ref, wo1_ref, wo2_ref, out_ref):
    r, d = hs_ref.shape
    k = nbt_ref.shape[1]
    hs = hs_ref[...]
    tw = tw_ref[...].reshape(1, 1, 16)
    tb = tb_ref[...].reshape(1, 1, 16)
    teq = _fast_cos(tb_ref[...])
    wq = wq_ref[...]
    q = (jnp.dot(hs, wq[:d], preferred_element_type=F32)
         + jnp.dot(teq, wq[d:], preferred_element_type=F32))
    dt = t_ref[...] - nbt_ref[...]
    ten = _fast_cos(dt[..., None] * tw + tb).reshape(r * k, 16)
    hn = hn_ref[...]
    nef = nef_ref[...].reshape(r * k, 16)
    wk, wv = wk_ref[...], wv_ref[...]
    kk = (jnp.dot(hn, wk[:d], preferred_element_type=F32)
          + jnp.dot(nef, wk[d:d + 16], preferred_element_type=F32)
          + jnp.dot(ten, wk[d + 16:], preferred_element_type=F32))
    vv = (jnp.dot(hn, wv[:d], preferred_element_type=F32)
          + jnp.dot(nef, wv[d:d + 16], preferred_element_type=F32)
          + jnp.dot(ten, wv[d + 16:], preferred_element_type=F32))
    logits = jnp.sum(q[:, None, :] * kk.reshape(r, k, d), axis=-1)
    logits = logits * (1.0 / jnp.sqrt(jnp.float32(d)))
    mx = jnp.max(logits, axis=-1, keepdims=True)
    ex = jnp.exp(logits - mx)
    attn = ex / jnp.sum(ex, axis=-1, keepdims=True)
    out = jnp.sum(attn[..., None] * vv.reshape(r, k, d), axis=1)
    wo1 = wo1_ref[...]
    hcat = (jnp.dot(out, wo1[:d], preferred_element_type=F32)
            + jnp.dot(hs, wo1[d:], preferred_element_type=F32))
    emb = jnp.dot(jax.nn.relu(hcat), wo2_ref[...],
                  preferred_element_type=F32)
    out_ref[...] = emb


def _tc_attn(h_src, h_nbr_flat, nbr_ef_flat, t, nbr_t, time_w, time_b,
             w_q, w_k, w_v, w_o1, w_o2):
    b, d = h_src.shape
    k = nbr_t.shape[1]
    blk = 512
    grid = b // blk
    return pl.pallas_call(
        _attn_body,
        grid=(grid,),
        in_specs=[
            pl.BlockSpec((blk, d), lambda i: (i, 0)),
            pl.BlockSpec((blk * k, d), lambda i: (i, 0)),
            pl.BlockSpec((blk, k, 16), lambda i: (i, 0, 0)),
            pl.BlockSpec((blk, k), lambda i: (i, 0)),
            pl.BlockSpec((blk, k), lambda i: (i, 0)),
            pl.BlockSpec((1, 16), lambda i: (0, 0)),
            pl.BlockSpec((1, 16), lambda i: (0, 0)),
            pl.BlockSpec(w_q.shape, lambda i: (0, 0)),
            pl.BlockSpec(w_k.shape, lambda i: (0, 0)),
            pl.BlockSpec(w_v.shape, lambda i: (0, 0)),
            pl.BlockSpec(w_o1.shape, lambda i: (0, 0)),
            pl.BlockSpec(w_o2.shape, lambda i: (0, 0)),
        ],
        out_specs=pl.BlockSpec((blk, d), lambda i: (i, 0)),
        out_shape=jax.ShapeDtypeStruct((b, d), F32),
    )(h_src, h_nbr_flat, nbr_ef_flat, jnp.broadcast_to(t[:, None], (b, k)),
      nbr_t, time_w.reshape(1, 16), time_b.reshape(1, 16), w_q, w_k, w_v,
      w_o1, w_o2)


def kernel(idx, t, src, dst, event_t, event_feat, nbr_idx, nbr_t,
           nbr_edge_feat, memory, node_feat, time_w, time_b, W_ih, b_ih,
           W_hh, b_hh, W_q, W_k, W_v, W_o1, W_o2):
    b, k = nbr_idx.shape
    n, d = memory.shape
    e = src.shape[0]
    accm = _sc_scatter_mem(src, dst, memory, n, d)
    accs = _sc_scatter_pay(src, dst, event_t, event_feat, time_w, time_b,
                           n, d)
    h_all = _tc_gru(accm, accs, memory, node_feat, W_ih, b_ih, W_hh, b_hh)
    h_src, h_nbr_flat = _sc_gather(h_all, idx, nbr_idx.reshape(-1))
    emb = _tc_attn(h_src, h_nbr_flat, nbr_edge_feat,
                   t, nbr_t, time_w, time_b, W_q, W_k, W_v, W_o1, W_o2)
    return emb


# speedup vs baseline: 6.5860x; 1.0381x over previous
import functools

import jax
import jax.numpy as jnp
from jax import lax
from jax.experimental import pallas as pl
from jax.experimental.pallas import tpu as pltpu
from jax.experimental.pallas import tpu_sc as plsc

NC, NS = 2, 16
CH = 128
CHM = 80
CHP = 128

F32 = jnp.float32

_COS_C = (0.999999999708885, -19.739208718081116, 64.9393881176947,
          -85.45664332355348, 60.24201925362776, -26.404267783730656,
          7.799566579651471, -1.4530463585373052)


def _fast_cos(x):
    u = x * 0.15915494309189535
    u = u - jnp.floor(u + 0.5)
    v = u * u
    p = jnp.float32(_COS_C[7])
    for cc in _COS_C[6::-1]:
        p = p * v + jnp.float32(cc)
    return p


def _sc_scatter_mem_body(src_hbm, dst_hbm, mem_hbm, zm_hbm, accm_out,
                         src_idx, dst_idx, rows_s, rows_d,
                         sem_s0, sem_s1, sem_d0, sem_d1, acc_m):
    n_pad = acc_m.shape[0]
    e = src_hbm.shape[0]
    c = lax.axis_index("c")
    s = lax.axis_index("s")
    rows_per = n_pad // NS
    r0 = s * rows_per
    pltpu.sync_copy(zm_hbm.at[pl.ds(r0, rows_per)], acc_m.at[pl.ds(r0, rows_per)])
    plsc.subcore_barrier()
    cpc = (e // CHM) // NC
    iters = (cpc + NS - 1) // NS
    sem_s = (sem_s0, sem_s1)
    sem_d = (sem_d0, sem_d1)

    def issue(i, slot):
        j = i * NS + s

        @pl.when(j < cpc)
        def _():
            base = (c * cpc + j) * CHM
            pltpu.sync_copy(src_hbm.at[pl.ds(base, CHM)], src_idx.at[slot])
            pltpu.sync_copy(dst_hbm.at[pl.ds(base, CHM)], dst_idx.at[slot])
            pltpu.async_copy(mem_hbm.at[dst_idx.at[slot]], rows_d.at[slot],
                             sem_d[slot])
            pltpu.async_copy(mem_hbm.at[src_idx.at[slot]], rows_s.at[slot],
                             sem_s[slot])

    def drain(i, slot):
        j = i * NS + s

        @pl.when(j < cpc)
        def _():
            pltpu.make_async_copy(mem_hbm.at[dst_idx.at[slot]],
                                  rows_d.at[slot], sem_d[slot]).wait()
            pltpu.sync_copy(rows_d.at[slot], acc_m.at[src_idx.at[slot]],
                            add=True)
            pltpu.make_async_copy(mem_hbm.at[src_idx.at[slot]],
                                  rows_s.at[slot], sem_s[slot]).wait()
            pltpu.sync_copy(rows_s.at[slot], acc_m.at[dst_idx.at[slot]],
                            add=True)

    issue(0, 0)

    def body(i2, carry):
        for b2 in (0, 1):
            i = i2 * 2 + b2
            issue(i + 1, 1 - b2)
            drain(i, b2)
        return carry

    lax.fori_loop(0, (iters + 1) // 2, body, 0)
    plsc.subcore_barrier()
    pltpu.sync_copy(acc_m.at[pl.ds(r0, rows_per)],
                    accm_out.at[c, pl.ds(r0, rows_per)])


def _fast_cos_sc(x):
    u = x * 0.15915494309189535
    y = u + 0.5
    tr = y.astype(jnp.int32).astype(F32)
    fl = jnp.where(tr > y, tr - 1.0, tr)
    u = u - fl
    v = u * u
    p = jnp.full_like(v, _COS_C[7])
    for cc in _COS_C[6::-1]:
        p = p * v + jnp.float32(cc)
    return p


def _sc_scatter_pay_body(src_hbm, dst_hbm, et_hbm, ef_hbm, tw_hbm, tb_hbm,
                         zm_hbm, accs_out,
                         src_idx, dst_idx, etb, efb, pay_buf, twb, tbb,
                         sem0, sem1, acc_s):
    n_pad = acc_s.shape[0]
    e = src_hbm.shape[0]
    c = lax.axis_index("c")
    s = lax.axis_index("s")
    rows_per = n_pad // NS
    r0 = s * rows_per
    pltpu.sync_copy(zm_hbm.at[pl.ds(r0, rows_per)], acc_s.at[pl.ds(r0, rows_per)])
    pltpu.sync_copy(tw_hbm, twb)
    pltpu.sync_copy(tb_hbm, tbb)
    one0 = jnp.where(lax.iota(jnp.int32, 16) == 0, 1.0, 0.0).astype(F32)
    zero16 = jnp.zeros((16,), F32)
    for slot in (0, 1):
        for i in range(CHP):
            pay_buf[slot, i, pl.ds(32, 16)] = one0
            for col in range(48, 128, 16):
                pay_buf[slot, i, pl.ds(col, 16)] = zero16
    plsc.subcore_barrier()
    cpc = (e // CHP) // NC
    iters = (cpc + NS - 1) // NS
    sems = (sem0, sem1)
    wv = twb[...]
    bv = tbb[...]

    def issue(i, slot):
        j = i * NS + s

        @pl.when(j < cpc)
        def _():
            base = (c * cpc + j) * CHP
            pltpu.sync_copy(src_hbm.at[pl.ds(base, CHP)], src_idx.at[slot])
            pltpu.sync_copy(dst_hbm.at[pl.ds(base, CHP)], dst_idx.at[slot])
            pltpu.sync_copy(et_hbm.at[pl.ds(base, CHP)], etb.at[slot])
            efrow = pl.multiple_of(base * 16 // 128, 8)
            pltpu.async_copy(ef_hbm.at[pl.ds(efrow, CHP * 16 // 128)],
                             efb.at[slot], sems[slot])

    def drain(i, slot):
        j = i * NS + s

        @pl.when(j < cpc)
        def _():
            pltpu.make_async_copy(
                ef_hbm.at[pl.ds(0, CHP * 16 // 128)], efb.at[slot],
                sems[slot]).wait()
            for g in range(CHP // 16):
                tv = etb[slot, pl.ds(g * 16, 16)]
                for l in range(16):
                    i_ev = g * 16 + l
                    te = _fast_cos_sc(tv[l] * wv + bv)
                    pay_buf[slot, i_ev, pl.ds(16, 16)] = te
                    pay_buf[slot, i_ev, pl.ds(0, 16)] = (
                        efb[slot, i_ev // 8, pl.ds(16 * (i_ev % 8), 16)])
            pltpu.sync_copy(pay_buf.at[slot], acc_s.at[src_idx.at[slot]],
                            add=True)
            pltpu.sync_copy(pay_buf.at[slot], acc_s.at[dst_idx.at[slot]],
                            add=True)

    issue(0, 0)

    def body(i2, carry):
        for b2 in (0, 1):
            i = i2 * 2 + b2
            issue(i + 1, 1 - b2)
            drain(i, b2)
        return carry

    lax.fori_loop(0, (iters + 1) // 2, body, 0)
    plsc.subcore_barrier()
    pltpu.sync_copy(acc_s.at[pl.ds(r0, rows_per)],
                    accs_out.at[c, pl.ds(r0, rows_per)])


def _sc_scatter_mem(src, dst, memory, n, d):
    n_pad = ((n + 8 * NS - 1) // (8 * NS)) * (8 * NS)
    mesh = plsc.VectorSubcoreMesh(core_axis_name="c", subcore_axis_name="s")
    zm = jnp.zeros((n_pad, d), F32)
    fn = functools.partial(
        pl.kernel,
        out_type=jax.ShapeDtypeStruct((NC, n_pad, d), F32),
        mesh=mesh,
        scratch_types=[
            pltpu.VMEM((2, CHM), jnp.int32),
            pltpu.VMEM((2, CHM), jnp.int32),
            pltpu.VMEM((2, CHM, d), F32),
            pltpu.VMEM((2, CHM, d), F32),
            pltpu.SemaphoreType.DMA,
            pltpu.SemaphoreType.DMA,
            pltpu.SemaphoreType.DMA,
            pltpu.SemaphoreType.DMA,
            pltpu.VMEM_SHARED((n_pad, d), F32),
        ],
    )(_sc_scatter_mem_body)
    return fn(src, dst, memory, zm)


def _sc_scatter_pay(src, dst, event_t, event_feat, time_w, time_b, n, d):
    n_pad = ((n + 8 * NS - 1) // (8 * NS)) * (8 * NS)
    e = src.shape[0]
    mesh = plsc.VectorSubcoreMesh(core_axis_name="c", subcore_axis_name="s")
    zm = jnp.zeros((n_pad, d), F32)
    ef128 = event_feat.reshape(e * 16 // 128, 128)
    fn = functools.partial(
        pl.kernel,
        out_type=jax.ShapeDtypeStruct((NC, n_pad, d), F32),
        mesh=mesh,
        scratch_types=[
            pltpu.VMEM((2, CHP), jnp.int32),
            pltpu.VMEM((2, CHP), jnp.int32),
            pltpu.VMEM((2, CHP), F32),
            pltpu.VMEM((2, CHP * 16 // 128, 128), F32),
            pltpu.VMEM((2, CHP, d), F32),
            pltpu.VMEM((16,), F32),
            pltpu.VMEM((16,), F32),
            pltpu.SemaphoreType.DMA,
            pltpu.SemaphoreType.DMA,
            pltpu.VMEM_SHARED((n_pad, d), F32),
        ],
    )(_sc_scatter_pay_body)
    return fn(src, dst, event_t, ef128, time_w, time_b, zm)


def _sc_gather_body(h_hbm, qidx_hbm, nidx_hbm, outq_hbm, outn_hbm,
                    idxb, rows, sem0, sem1):
    c = lax.axis_index("c")
    s = lax.axis_index("s")
    w = s * NC + c
    nw = NC * NS
    iters_n = (nidx_hbm.shape[0] // CH) // nw
    sems = (sem0, sem1)

    def issue(i, slot):
        @pl.when(i <= iters_n)
        def _():
            qbase = w * CH
            nbase = ((i - 1) * nw + w) * CH
            @pl.when(i == 0)
            def _():
                pltpu.sync_copy(qidx_hbm.at[pl.ds(qbase, CH)], idxb.at[slot])
            @pl.when(i > 0)
            def _():
                pltpu.sync_copy(nidx_hbm.at[pl.ds(nbase, CH)], idxb.at[slot])
            pltpu.async_copy(h_hbm.at[idxb.at[slot]], rows.at[slot],
                             sems[slot])

    def drain(i, slot):
        @pl.when(i <= iters_n)
        def _():
            pltpu.make_async_copy(h_hbm.at[idxb.at[slot]], rows.at[slot],
                                  sems[slot]).wait()
            @pl.when(i == 0)
            def _():
                pltpu.sync_copy(rows.at[slot], outq_hbm.at[pl.ds(w * CH, CH)])
            @pl.when(i > 0)
            def _():
                nbase = ((i - 1) * nw + w) * CH
                pltpu.sync_copy(rows.at[slot], outn_hbm.at[pl.ds(nbase, CH)])

    issue(0, 0)

    def body(i2, carry):
        for b2 in (0, 1):
            i = i2 * 2 + b2
            issue(i + 1, 1 - b2)
            drain(i, b2)
        return carry

    lax.fori_loop(0, (iters_n + 2) // 2, body, 0)


def _sc_gather(h_all, qidx, nidx):
    n, d = h_all.shape
    mesh = plsc.VectorSubcoreMesh(core_axis_name="c", subcore_axis_name="s")
    fn = functools.partial(
        pl.kernel,
        out_type=[jax.ShapeDtypeStruct((qidx.shape[0], d), F32),
                  jax.ShapeDtypeStruct((nidx.shape[0], d), F32)],
        mesh=mesh,
        scratch_types=[
            pltpu.VMEM((2, CH), jnp.int32),
            pltpu.VMEM((2, CH, d), F32),
            pltpu.SemaphoreType.DMA,
            pltpu.SemaphoreType.DMA,
        ],
    )(_sc_gather_body)
    return fn(h_all, qidx, nidx)


def _gru_body(accm_ref, accs_ref, mem_ref, nf_ref, wih_ref, bih_ref, whh_ref,
              bhh_ref, out_ref):
    am = accm_ref[0] + accm_ref[1]
    asml = accs_ref[0] + accs_ref[1]
    cnt = asml[:, 32:33]
    inv = 1.0 / jnp.maximum(cnt, 1.0)
    m = mem_ref[...]
    mm = jnp.concatenate([m, am * inv, asml[:, :32] * inv], axis=1)
    gi = jnp.dot(mm, wih_ref[...], preferred_element_type=F32) + bih_ref[...]
    gh = jnp.dot(m, whh_ref[...], preferred_element_type=F32) + bhh_ref[...]
    d = m.shape[1]
    r = jax.nn.sigmoid(gi[:, :d] + gh[:, :d])
    z = jax.nn.sigmoid(gi[:, d:2 * d] + gh[:, d:2 * d])
    nn = jnp.tanh(gi[:, 2 * d:] + r * gh[:, 2 * d:])
    new_mem = (1.0 - z) * nn + z * m
    out_ref[...] = jnp.where(cnt > 0, new_mem, m) + nf_ref[...]


def _tc_gru(accm, accs, memory, node_feat, w_ih, b_ih, w_hh, b_hh):
    n, d = memory.shape
    blk = 2000
    grid = n // blk
    return pl.pallas_call(
        _gru_body,
        grid=(grid,),
        in_specs=[
            pl.BlockSpec((NC, blk, d), lambda i: (0, i, 0)),
            pl.BlockSpec((NC, blk, d), lambda i: (0, i, 0)),
            pl.BlockSpec((blk, d), lambda i: (i, 0)),
            pl.BlockSpec((blk, d), lambda i: (i, 0)),
            pl.BlockSpec(w_ih.shape, lambda i: (0, 0)),
            pl.BlockSpec((1, 3 * d), lambda i: (0, 0)),
            pl.BlockSpec(w_hh.shape, lambda i: (0, 0)),
            pl.BlockSpec((1, 3 * d), lambda i: (0, 0)),
        ],
        out_specs=pl.BlockSpec((blk, d), lambda i: (i, 0)),
        out_shape=jax.ShapeDtypeStruct((n, d), F32),
    )(accm, accs, memory, node_feat, w_ih, b_ih.reshape(1, -1), w_hh,
      b_hh.reshape(1, -1))


def _attn_body(hs_ref, hn_ref, nef_ref, t_ref, nbt_ref, tw_ref, tb_ref,
               wq_ref, wk_ref, wv_ref, wo1_ref, wo2_ref, out_ref):
    r, d = hs_ref.shape
    k = nbt_ref.shape[1]
    hs = hs_ref[...]
    tw = tw_ref[...].reshape(1, 1, 16)
    tb = tb_ref[...].reshape(1, 1, 16)
    teq = _fast_cos(tb_ref[...])
    wq = wq_ref[...]
    q = (jnp.dot(hs, wq[:d], preferred_element_type=F32)
         + jnp.dot(teq, wq[d:], preferred_element_type=F32))
    dt = t_ref[...] - nbt_ref[...]
    ten = _fast_cos(dt[..., None] * tw + tb).reshape(r * k, 16)
    hn = hn_ref[...]
    nef = nef_ref[...].reshape(r * k, 16)
    wk, wv = wk_ref[...], wv_ref[...]
    kk = (jnp.dot(hn, wk[:d], preferred_element_type=F32)
          + jnp.dot(nef, wk[d:d + 16], preferred_element_type=F32)
          + jnp.dot(ten, wk[d + 16:], preferred_element_type=F32))
    vv = (jnp.dot(hn, wv[:d], preferred_element_type=F32)
          + jnp.dot(nef, wv[d:d + 16], preferred_element_type=F32)
          + jnp.dot(ten, wv[d + 16:], preferred_element_type=F32))
    logits = jnp.sum(q[:, None, :] * kk.reshape(r, k, d), axis=-1)
    logits = logits * (1.0 / jnp.sqrt(jnp.float32(d)))
    mx = jnp.max(logits, axis=-1, keepdims=True)
    ex = jnp.exp(logits - mx)
    attn = ex / jnp.sum(ex, axis=-1, keepdims=True)
    out = jnp.sum(attn[..., None] * vv.reshape(r, k, d), axis=1)
    wo1 = wo1_ref[...]
    hcat = (jnp.dot(out, wo1[:d], preferred_element_type=F32)
            + jnp.dot(hs, wo1[d:], preferred_element_type=F32))
    emb = jnp.dot(jax.nn.relu(hcat), wo2_ref[...],
                  preferred_element_type=F32)
    out_ref[...] = emb


def _tc_attn(h_src, h_nbr_flat, nbr_ef_flat, t, nbr_t, time_w, time_b,
             w_q, w_k, w_v, w_o1, w_o2):
    b, d = h_src.shape
    k = nbr_t.shape[1]
    blk = 512
    grid = b // blk
    return pl.pallas_call(
        _attn_body,
        grid=(grid,),
        in_specs=[
            pl.BlockSpec((blk, d), lambda i: (i, 0)),
            pl.BlockSpec((blk * k, d), lambda i: (i, 0)),
            pl.BlockSpec((blk, k, 16), lambda i: (i, 0, 0)),
            pl.BlockSpec((blk, k), lambda i: (i, 0)),
            pl.BlockSpec((blk, k), lambda i: (i, 0)),
            pl.BlockSpec((1, 16), lambda i: (0, 0)),
            pl.BlockSpec((1, 16), lambda i: (0, 0)),
            pl.BlockSpec(w_q.shape, lambda i: (0, 0)),
            pl.BlockSpec(w_k.shape, lambda i: (0, 0)),
            pl.BlockSpec(w_v.shape, lambda i: (0, 0)),
            pl.BlockSpec(w_o1.shape, lambda i: (0, 0)),
            pl.BlockSpec(w_o2.shape, lambda i: (0, 0)),
        ],
        out_specs=pl.BlockSpec((blk, d), lambda i: (i, 0)),
        out_shape=jax.ShapeDtypeStruct((b, d), F32),
    )(h_src, h_nbr_flat, nbr_ef_flat, jnp.broadcast_to(t[:, None], (b, k)),
      nbr_t, time_w.reshape(1, 16), time_b.reshape(1, 16), w_q, w_k, w_v,
      w_o1, w_o2)


def kernel(idx, t, src, dst, event_t, event_feat, nbr_idx, nbr_t,
           nbr_edge_feat, memory, node_feat, time_w, time_b, W_ih, b_ih,
           W_hh, b_hh, W_q, W_k, W_v, W_o1, W_o2):
    b, k = nbr_idx.shape
    n, d = memory.shape
    e = src.shape[0]
    accm = _sc_scatter_mem(src, dst, memory, n, d)
    accs = _sc_scatter_pay(src, dst, event_t, event_feat, time_w, time_b,
                           n, d)
    h_all = _tc_gru(accm, accs, memory, node_feat, W_ih, b_ih, W_hh, b_hh)
    h_src, h_nbr_flat = _sc_gather(h_all, idx, nbr_idx.reshape(-1))
    emb = _tc_attn(h_src, h_nbr_flat, nbr_edge_feat,
                   t, nbr_t, time_w, time_b, W_q, W_k, W_v, W_o1, W_o2)
    return emb
